# baseline (speedup 1.0000x reference)
import functools

import jax
import jax.numpy as jnp
from jax import lax
from jax.experimental import pallas as pl
from jax.experimental.pallas import tpu as pltpu
from jax.experimental.pallas import tpu_sc as plsc

N = 10000
E = 320000
H = 128
C = 2
THETAS = ((3.0, -3.0, 0.75), (0.0, 3.0, -1.5), (0.0, 0.0, 0.75))

NC, NS, L = 2, 16, 16
NP = 10240
RPT = 80
RP = NC * NS * RPT
EP = RP * 128
NPT = NP // NS
BN = 512
F32 = jnp.float32


def _sc_mesh():
    return plsc.VectorSubcoreMesh(core_axis_name="c", subcore_axis_name="s",
                                  num_cores=NC, num_subcores=NS)


def _seg_scalar_body(vals_hbm, dst_hbm, zeros_hbm, out_hbm, idx_d, val_v, acc, sem):
    del sem
    cid = lax.axis_index("c")
    sid = lax.axis_index("s")
    pltpu.sync_copy(zeros_hbm.at[pl.ds(sid * NPT, NPT)], acc.at[pl.ds(sid * NPT, NPT)])
    plsc.subcore_barrier()
    base = (cid * NS + sid) * RPT

    def chunk(j, carry):
        r = base + j
        pltpu.sync_copy(dst_hbm.at[r], idx_d)
        pltpu.sync_copy(vals_hbm.at[r], val_v)
        pltpu.sync_copy(val_v, acc.at[idx_d], add=True)
        return carry

    lax.fori_loop(0, RPT, chunk, 0)
    plsc.subcore_barrier()
    pltpu.sync_copy(acc.at[pl.ds(sid * NPT, NPT)], out_hbm.at[cid, pl.ds(sid * NPT, NPT)])


@functools.cache
def _seg_scalar_kernel():
    return pl.kernel(
        _seg_scalar_body,
        out_type=jax.ShapeDtypeStruct((NC, NP), F32),
        mesh=_sc_mesh(),
        scratch_types=[
            pltpu.VMEM((128,), jnp.int32),
            pltpu.VMEM((128,), F32),
            pltpu.VMEM_SHARED((NP,), F32),
            pltpu.SemaphoreType.DMA,
        ],
    )


def _seg_scalar(vals2d, dst2d, z1d):
    return _seg_scalar_kernel()(vals2d, dst2d, z1d)


def _seg_rows_body(g_hbm, src_hbm, dst_hbm, zeros_hbm, out_hbm,
                   ixs0, ixs1, ixd, rows0, rows1, sg0, sg1, acc):
    cid = lax.axis_index("c")
    sid = lax.axis_index("s")
    pltpu.sync_copy(zeros_hbm.at[pl.ds(sid * NPT, NPT)], acc.at[pl.ds(sid * NPT, NPT)])
    base = (cid * NS + sid) * RPT
    plsc.subcore_barrier()

    pltpu.sync_copy(src_hbm.at[base], ixs0)
    pltpu.async_copy(g_hbm.at[ixs0], rows0, sg0)

    def it(j, carry):
        def step(ixa, ixb, rb, ob, sg, sg_n):
            @pl.when(j + 1 < RPT)
            def _():
                pltpu.sync_copy(src_hbm.at[base + j + 1], ixb)
                pltpu.async_copy(g_hbm.at[ixb], ob, sg_n)

            pltpu.sync_copy(dst_hbm.at[base + j], ixd)
            pltpu.make_async_copy(g_hbm.at[ixa], rb, sg).wait()
            pltpu.sync_copy(rb, acc.at[ixd], add=True)

        @pl.when(j % 2 == 0)
        def _():
            step(ixs0, ixs1, rows0, rows1, sg0, sg1)

        @pl.when(j % 2 == 1)
        def _():
            step(ixs1, ixs0, rows1, rows0, sg1, sg0)

        return carry

    lax.fori_loop(0, RPT, it, 0)
    plsc.subcore_barrier()
    pltpu.sync_copy(acc.at[pl.ds(sid * NPT, NPT)], out_hbm.at[cid, pl.ds(sid * NPT, NPT)])


@functools.cache
def _seg_rows_kernel():
    return pl.kernel(
        _seg_rows_body,
        out_type=jax.ShapeDtypeStruct((NC, NP, H), F32),
        mesh=_sc_mesh(),
        scratch_types=[
            pltpu.VMEM((128,), jnp.int32),
            pltpu.VMEM((128,), jnp.int32),
            pltpu.VMEM((128,), jnp.int32),
            pltpu.VMEM((128, H), F32),
            pltpu.VMEM((128, H), F32),
            pltpu.SemaphoreType.DMA,
            pltpu.SemaphoreType.DMA,
            pltpu.VMEM_SHARED((NP, H), F32),
        ],
    )


def _seg_rows(g, src2d, dst2d, zrows):
    return _seg_rows_kernel()(g, src2d, dst2d, zrows)


def _gather_pair_body(g1_hbm, g2_hbm, src_hbm, dst_hbm, a_hbm, b_hbm, idx_s, idx_d, rows, rows2, sem):
    cid = lax.axis_index("c")
    sid = lax.axis_index("s")
    base = (cid * NS + sid) * RPT

    def chunk(j, carry):
        r = base + j
        pltpu.sync_copy(src_hbm.at[r], idx_s)
        pltpu.sync_copy(dst_hbm.at[r], idx_d)
        pltpu.async_copy(g1_hbm.at[idx_s], rows, sem).wait()
        pltpu.sync_copy(rows, a_hbm.at[r])
        pltpu.async_copy(g2_hbm.at[idx_d], rows2, sem).wait()
        pltpu.sync_copy(rows2, b_hbm.at[r])
        return carry

    lax.fori_loop(0, RPT, chunk, 0)


@functools.cache
def _gather_pair_kernel():
    return pl.kernel(
        _gather_pair_body,
        out_type=(
            jax.ShapeDtypeStruct((RP, 128, H), F32),
            jax.ShapeDtypeStruct((RP, 128, H), F32),
        ),
        mesh=_sc_mesh(),
        scratch_types=[
            pltpu.VMEM((128,), jnp.int32),
            pltpu.VMEM((128,), jnp.int32),
            pltpu.VMEM((128, H), F32),
            pltpu.VMEM((128, H), F32),
            pltpu.SemaphoreType.DMA,
        ],
    )


def _gather_pair(G1, G2, src2d, dst2d):
    return _gather_pair_kernel()(G1, G2, src2d, dst2d)


def _seg_masked_body(g_hbm, src_hbm, dst_hbm, seb_hbm, zeros_hbm, out_hbm,
                     idx_s, idx_d, seb_v, rows, acc, sem):
    cid = lax.axis_index("c")
    sid = lax.axis_index("s")
    pltpu.sync_copy(zeros_hbm.at[pl.ds(sid * NPT, NPT)], acc.at[pl.ds(sid * NPT, NPT)])
    plsc.subcore_barrier()
    base = (cid * NS + sid) * RPT

    def chunk(j, carry):
        r = base + j
        pltpu.sync_copy(src_hbm.at[r], idx_s)
        pltpu.sync_copy(dst_hbm.at[r], idx_d)
        pltpu.sync_copy(seb_hbm.at[pl.ds(r * 128, 128)], seb_v)
        pltpu.async_copy(g_hbm.at[idx_s], rows, sem).wait()

        def ebody(e, carry2):
            mk = seb_v[e]
            for k in range(8):
                sl = pl.ds(16 * k, 16)
                rows[e, sl] = rows[e, sl] * mk
            return carry2

        lax.fori_loop(0, 128, ebody, 0)
        pltpu.sync_copy(rows, acc.at[idx_d], add=True)
        return carry

    lax.fori_loop(0, RPT, chunk, 0)
    plsc.subcore_barrier()
    pltpu.sync_copy(acc.at[pl.ds(sid * NPT, NPT)], out_hbm.at[cid, pl.ds(sid * NPT, NPT)])


@functools.cache
def _seg_masked_kernel():
    return pl.kernel(
        _seg_masked_body,
        out_type=jax.ShapeDtypeStruct((NC, NP, H), F32),
        mesh=_sc_mesh(),
        scratch_types=[
            pltpu.VMEM((128,), jnp.int32),
            pltpu.VMEM((128,), jnp.int32),
            pltpu.VMEM((128, L), F32),
            pltpu.VMEM((128, H), F32),
            pltpu.VMEM_SHARED((NP, H), F32),
            pltpu.SemaphoreType.DMA,
        ],
    )


def _seg_masked(g, src2d, dst2d, seb, zrows):
    return _seg_masked_kernel()(g, src2d, dst2d, seb, zrows)


_NBLK = NP // BN
_EBLK = EP // BN


def _mlp_body(x_ref, w1_ref, b1_ref, w2_ref, b2_ref, o_ref):
    h = jnp.maximum(jnp.dot(x_ref[...], w1_ref[...], preferred_element_type=F32) + b1_ref[...], 0.0)
    o_ref[...] = jnp.maximum(jnp.dot(h, w2_ref[...], preferred_element_type=F32) + b2_ref[...], 0.0)


def _tc_mlp(x, w1, b1, w2, b2):
    return pl.pallas_call(
        _mlp_body,
        grid=(_NBLK,),
        in_specs=[
            pl.BlockSpec((BN, H), lambda i: (i, 0)),
            pl.BlockSpec((H, H), lambda i: (0, 0)),
            pl.BlockSpec((1, H), lambda i: (0, 0)),
            pl.BlockSpec((H, H), lambda i: (0, 0)),
            pl.BlockSpec((1, H), lambda i: (0, 0)),
        ],
        out_specs=pl.BlockSpec((BN, H), lambda i: (i, 0)),
        out_shape=jax.ShapeDtypeStruct((NP, H), F32),
    )(x, w1, b1, w2, b2)


def _scale_body(h_ref, degp_ref, g_ref, dinv_ref):
    deg = degp_ref[0, :] + degp_ref[1, :]
    dinv = lax.rsqrt(jnp.maximum(deg, 1.0))
    dinvf = jnp.broadcast_to(dinv[:, None], (BN, H))
    dinv_ref[...] = dinvf
    g_ref[...] = h_ref[...] * dinvf


def _tc_scale(h, degp):
    return pl.pallas_call(
        _scale_body,
        grid=(_NBLK,),
        in_specs=[
            pl.BlockSpec((BN, H), lambda i: (i, 0)),
            pl.BlockSpec((2, BN), lambda i: (0, i)),
        ],
        out_specs=[
            pl.BlockSpec((BN, H), lambda i: (i, 0)),
            pl.BlockSpec((BN, H), lambda i: (i, 0)),
        ],
        out_shape=[
            jax.ShapeDtypeStruct((NP, H), F32),
            jax.ShapeDtypeStruct((NP, H), F32),
        ],
    )(h, degp)


def _fstep_body(base_ref, p_ref, dinv_ref, f_ref, g_ref):
    ps = p_ref[0] + p_ref[1]
    dinv = dinv_ref[...]
    f = base_ref[...] - ps * dinv
    f_ref[...] = f
    g_ref[...] = f * dinv


def _tc_fstep(base, p, dinvf):
    return pl.pallas_call(
        _fstep_body,
        grid=(_NBLK,),
        in_specs=[
            pl.BlockSpec((BN, H), lambda i: (i, 0)),
            pl.BlockSpec((2, BN, H), lambda i: (0, i, 0)),
            pl.BlockSpec((BN, H), lambda i: (i, 0)),
        ],
        out_specs=[
            pl.BlockSpec((BN, H), lambda i: (i, 0)),
            pl.BlockSpec((BN, H), lambda i: (i, 0)),
        ],
        out_shape=[
            jax.ShapeDtypeStruct((NP, H), F32),
            jax.ShapeDtypeStruct((NP, H), F32),
        ],
    )(base, p, dinvf)


def _big_body(h_ref, f1_ref, p_ref, dinv_ref, b0_ref, b1_ref, b2_ref, bias_ref,
              wls_ref, wld_ref, wlmu_ref, bl1_ref,
              f2_ref, g1_ref, g2_ref, c_ref):
    i = pl.program_id(0)
    dinv = dinv_ref[...]
    f2 = f1_ref[...] - (p_ref[0] + p_ref[1]) * dinv
    f2_ref[...] = f2
    h1 = (jnp.dot(h_ref[...], b0_ref[...], preferred_element_type=F32)
          + jnp.dot(f1_ref[...], b1_ref[...], preferred_element_type=F32)
          + jnp.dot(f2, b2_ref[...], preferred_element_type=F32)
          + bias_ref[...])
    g1_ref[...] = jnp.dot(h1, wls_ref[...], preferred_element_type=F32)
    g2_ref[...] = jnp.dot(h1, wld_ref[...], preferred_element_type=F32)
    rowid = i * BN + lax.broadcasted_iota(jnp.int32, (BN, 1), 0)
    h1m = jnp.where(rowid < N, h1, 0.0)
    part = jnp.sum(h1m.reshape(BN // 8, 8, H), axis=0)

    @pl.when(i == 0)
    def _():
        c_ref[...] = jnp.zeros((8, H), F32)

    acc = c_ref[...] + part
    c_ref[...] = acc

    @pl.when(i == _NBLK - 1)
    def _():
        mu = jnp.sum(acc, axis=0, keepdims=True) / float(N)
        cvec = bl1_ref[...] - jnp.dot(mu, wlmu_ref[...], preferred_element_type=F32)
        c_ref[...] = jnp.broadcast_to(cvec, (8, H))


def _tc_big(h, f1, p2, dinvf, b0, b1, b2, bias3, wls, wld, wlmu, bl1):
    full = lambda shape: pl.BlockSpec(shape, lambda i: tuple(0 for _ in shape))
    return pl.pallas_call(
        _big_body,
        grid=(_NBLK,),
        in_specs=[
            pl.BlockSpec((BN, H), lambda i: (i, 0)),
            pl.BlockSpec((BN, H), lambda i: (i, 0)),
            pl.BlockSpec((2, BN, H), lambda i: (0, i, 0)),
            pl.BlockSpec((BN, H), lambda i: (i, 0)),
            full((H, H)), full((H, H)), full((H, H)), full((1, H)),
            full((H, H)), full((H, H)), full((H, H)), full((1, H)),
        ],
        out_specs=[
            pl.BlockSpec((BN, H), lambda i: (i, 0)),
            pl.BlockSpec((BN, H), lambda i: (i, 0)),
            pl.BlockSpec((BN, H), lambda i: (i, 0)),
            pl.BlockSpec((8, H), lambda i: (0, 0)),
        ],
        out_shape=[
            jax.ShapeDtypeStruct((NP, H), F32),
            jax.ShapeDtypeStruct((NP, H), F32),
            jax.ShapeDtypeStruct((NP, H), F32),
            jax.ShapeDtypeStruct((8, H), F32),
        ],
    )(h, f1, p2, dinvf, b0, b1, b2, bias3, wls, wld, wlmu, bl1)


BE = 1024


def _edge_body(a_ref, b_ref, c_ref, leps_ref, wl2_ref, bl2_ref, m2_ref, seb_ref):
    i = pl.program_id(0)
    cvec = c_ref[0:1, :]
    epre = a_ref[...] + b_ref[...] + cvec
    ef = jnp.where(epre > 0, epre, 0.01 * epre)
    s = jnp.sum(ef * wl2_ref[...], axis=1) + bl2_ref[0, 0]
    ef2 = jax.nn.sigmoid(s)
    leps = leps_ref[...].reshape(BE)
    m2 = jax.nn.sigmoid((leps + ef2) * 10.0)
    eid = i * BE + lax.iota(jnp.int32, BE)
    m2 = jnp.where(eid < E, m2, 0.0)
    m2_ref[...] = m2.reshape(BE // 128, 128)
    seb_ref[...] = jnp.broadcast_to(m2[:, None], (BE, L))


def _tc_edge(a, b, cmat, leps2d, wl2row, bl2m):
    return pl.pallas_call(
        _edge_body,
        grid=(EP // BE,),
        in_specs=[
            pl.BlockSpec((BE, H), lambda i: (i, 0)),
            pl.BlockSpec((BE, H), lambda i: (i, 0)),
            pl.BlockSpec((8, H), lambda i: (0, 0)),
            pl.BlockSpec((BE // 128, 128), lambda i: (i, 0)),
            pl.BlockSpec((1, H), lambda i: (0, 0)),
            pl.BlockSpec((1, 1), lambda i: (0, 0)),
        ],
        out_specs=[
            pl.BlockSpec((BE // 128, 128), lambda i: (i, 0)),
            pl.BlockSpec((BE, L), lambda i: (i, 0)),
        ],
        out_shape=[
            jax.ShapeDtypeStruct((RP, 128), F32),
            jax.ShapeDtypeStruct((EP, L), F32),
        ],
    )(a, b, cmat, leps2d, wl2row, bl2m)


def _final_body(h_ref, f1_ref, f2_ref, f1p_ref, q_ref, dinv_ref,
                e0_ref, c1_ref, c2_ref, d1_ref, d2_ref, b31_ref, w4_ref, b4_ref,
                o_ref):
    f2p = f1p_ref[...] - (q_ref[0] + q_ref[1]) * dinv_ref[...]
    h2 = (jnp.dot(h_ref[...], e0_ref[...], preferred_element_type=F32)
          + jnp.dot(f1_ref[...], c1_ref[...], preferred_element_type=F32)
          + jnp.dot(f2_ref[...], c2_ref[...], preferred_element_type=F32)
          + jnp.dot(f1p_ref[...], d1_ref[...], preferred_element_type=F32)
          + jnp.dot(f2p, d2_ref[...], preferred_element_type=F32)
          + b31_ref[...])
    h2 = jnp.maximum(h2, 0.0)
    o_ref[...] = jnp.dot(h2, w4_ref[...], preferred_element_type=F32) + b4_ref[...]


def _tc_final(h, f1, f2, f1p, q2, dinvf, e0, c1, c2, d1, d2, b31, w4p, b4p):
    full = lambda shape: pl.BlockSpec(shape, lambda i: tuple(0 for _ in shape))
    return pl.pallas_call(
        _final_body,
        grid=(_NBLK,),
        in_specs=[
            pl.BlockSpec((BN, H), lambda i: (i, 0)),
            pl.BlockSpec((BN, H), lambda i: (i, 0)),
            pl.BlockSpec((BN, H), lambda i: (i, 0)),
            pl.BlockSpec((BN, H), lambda i: (i, 0)),
            pl.BlockSpec((2, BN, H), lambda i: (0, i, 0)),
            pl.BlockSpec((BN, H), lambda i: (i, 0)),
            full((H, H)), full((H, H)), full((H, H)), full((H, H)), full((H, H)),
            full((1, H)), full((H, H)), full((1, H)),
        ],
        out_specs=pl.BlockSpec((BN, H), lambda i: (i, 0)),
        out_shape=jax.ShapeDtypeStruct((NP, H), F32),
    )(h, f1, f2, f1p, q2, dinvf, e0, c1, c2, d1, d2, b31, w4p, b4p)


def kernel(in_feat, src, dst, W1, b1, W2, b2, W3, b3, W3_1, b3_1, W4, b4, Wl1, bl1, Wl2, bl2):
    th = jnp.asarray(THETAS, F32)
    B = [sum(th[t, k] * W3[t * H:(t + 1) * H] for t in range(3)) for k in range(3)]
    Cc = [sum(th[t, k] * W3_1[t * H:(t + 1) * H] for t in range(3)) for k in range(3)]
    D = [sum(th[t, k] * W3_1[(3 + t) * H:(4 + t) * H] for t in range(3)) for k in range(3)]
    Wla, Wlb, Wlc = Wl1[0:H], Wl1[H:2 * H], Wl1[2 * H:3 * H]
    wls, wld, wlmu = Wla + Wlb, Wlc - Wla, Wlb + Wlc

    x_p = jnp.zeros((NP, H), F32).at[:N].set(in_feat)
    src2d = jnp.concatenate([src, jnp.zeros((EP - E,), jnp.int32)]).reshape(RP, 128)
    dst2d = jnp.concatenate([dst, jnp.full((EP - E,), N, jnp.int32)]).reshape(RP, 128)

    bias_u = 0.0001
    eps = (bias_u - (1.0 - bias_u)) * jax.random.uniform(jax.random.key(123), (E,), F32) + (1.0 - bias_u)
    leps = jnp.log(eps) - jnp.log(1.0 - eps)
    leps2d = jnp.concatenate([leps, jnp.zeros((EP - E,), F32)]).reshape(RP, 128)

    zrows = jnp.zeros((NP, H), F32)
    z1d = jnp.zeros((NP,), F32)
    ones2d = jnp.ones((RP, 128), F32)

    b3r = b3.reshape(1, H)
    b31r = b3_1.reshape(1, H)
    bl1r = bl1.reshape(1, H)
    wl2row = Wl2.reshape(1, H)
    bl2m = bl2.reshape(1, 1)
    w4p = jnp.zeros((H, H), F32).at[:, :C].set(W4)
    b4p = jnp.zeros((1, H), F32).at[0, :C].set(b4)

    h = _tc_mlp(x_p, W1, b1.reshape(1, H), W2, b2.reshape(1, H))
    deg1p = _seg_scalar(ones2d, dst2d, z1d)

    g0, dinv1f = _tc_scale(h, deg1p)
    p1 = _seg_rows(g0, src2d, dst2d, zrows)
    f1, g1 = _tc_fstep(h, p1, dinv1f)
    p2 = _seg_rows(g1, src2d, dst2d, zrows)
    f2, G1, G2, cmat = _tc_big(h, f1, p2, dinv1f, B[0], B[1], B[2], b3r,
                               wls, wld, wlmu, bl1r)

    a3, b3d = _gather_pair(G1, G2, src2d, dst2d)
    m2, seb = _tc_edge(a3.reshape(EP, H), b3d.reshape(EP, H), cmat, leps2d, wl2row, bl2m)

    deg2p = _seg_scalar(m2, dst2d, z1d)
    g0p, dinv2f = _tc_scale(h, deg2p)

    q1 = _seg_masked(g0p, src2d, dst2d, seb, zrows)
    f1p, g1p = _tc_fstep(h, q1, dinv2f)
    q2 = _seg_masked(g1p, src2d, dst2d, seb, zrows)

    out = _tc_final(h, f1, f2, f1p, q2, dinv2f,
                    Cc[0] + D[0], Cc[1], Cc[2], D[1], D[2], b31r, w4p, b4p)
    return out[:N, :C]


# speedup vs baseline: 2.0383x; 2.0383x over previous
import functools

import jax
import jax.numpy as jnp
from jax import lax
from jax.experimental import pallas as pl
from jax.experimental.pallas import tpu as pltpu
from jax.experimental.pallas import tpu_sc as plsc

N = 10000
E = 320000
H = 128
C = 2
THETAS = ((3.0, -3.0, 0.75), (0.0, 3.0, -1.5), (0.0, 0.0, 0.75))

NC, NS, L = 2, 16, 16
NP = 10240
RPT = 80
RP = NC * NS * RPT
EP = RP * 128
NPT = NP // NS
BN = 512
F32 = jnp.float32


def _sc_mesh():
    return plsc.VectorSubcoreMesh(core_axis_name="c", subcore_axis_name="s",
                                  num_cores=NC, num_subcores=NS)


def _seg_scalar_body(vals_hbm, dst_hbm, zeros_hbm, out_hbm, idx_d, val_v, acc, sem):
    del sem
    cid = lax.axis_index("c")
    sid = lax.axis_index("s")
    pltpu.sync_copy(zeros_hbm.at[pl.ds(sid * NPT, NPT)], acc.at[pl.ds(sid * NPT, NPT)])
    plsc.subcore_barrier()
    base = (cid * NS + sid) * RPT

    def chunk(j, carry):
        r = base + j
        pltpu.sync_copy(dst_hbm.at[r], idx_d)
        pltpu.sync_copy(vals_hbm.at[r], val_v)
        pltpu.sync_copy(val_v, acc.at[idx_d], add=True)
        return carry

    lax.fori_loop(0, RPT, chunk, 0)
    plsc.subcore_barrier()
    pltpu.sync_copy(acc.at[pl.ds(sid * NPT, NPT)], out_hbm.at[cid, pl.ds(sid * NPT, NPT)])


@functools.cache
def _seg_scalar_kernel():
    return pl.kernel(
        _seg_scalar_body,
        out_type=jax.ShapeDtypeStruct((NC, NP), F32),
        mesh=_sc_mesh(),
        scratch_types=[
            pltpu.VMEM((128,), jnp.int32),
            pltpu.VMEM((128,), F32),
            pltpu.VMEM_SHARED((NP,), F32),
            pltpu.SemaphoreType.DMA,
        ],
    )


def _seg_scalar(vals2d, dst2d, z1d):
    return _seg_scalar_kernel()(vals2d, dst2d, z1d)


def _seg_rows_body(g_hbm, src_hbm, dst_hbm, zeros_hbm, out_hbm,
                   ixs0, ixs1, ixd, rows0, rows1, sg0, sg1, acc):
    cid = lax.axis_index("c")
    sid = lax.axis_index("s")
    pltpu.sync_copy(zeros_hbm.at[pl.ds(sid * NPT, NPT)], acc.at[pl.ds(sid * NPT, NPT)])
    base = (cid * NS + sid) * RPT
    plsc.subcore_barrier()

    pltpu.sync_copy(src_hbm.at[base], ixs0)
    pltpu.async_copy(g_hbm.at[ixs0], rows0, sg0)

    def it(j, carry):
        def step(ixa, ixb, rb, ob, sg, sg_n):
            @pl.when(j + 1 < RPT)
            def _():
                pltpu.sync_copy(src_hbm.at[base + j + 1], ixb)
                pltpu.async_copy(g_hbm.at[ixb], ob, sg_n)

            pltpu.sync_copy(dst_hbm.at[base + j], ixd)
            pltpu.make_async_copy(g_hbm.at[ixa], rb, sg).wait()
            pltpu.sync_copy(rb, acc.at[ixd], add=True)

        @pl.when(j % 2 == 0)
        def _():
            step(ixs0, ixs1, rows0, rows1, sg0, sg1)

        @pl.when(j % 2 == 1)
        def _():
            step(ixs1, ixs0, rows1, rows0, sg1, sg0)

        return carry

    lax.fori_loop(0, RPT, it, 0)
    plsc.subcore_barrier()
    pltpu.sync_copy(acc.at[pl.ds(sid * NPT, NPT)], out_hbm.at[cid, pl.ds(sid * NPT, NPT)])


@functools.cache
def _seg_rows_kernel():
    return pl.kernel(
        _seg_rows_body,
        out_type=jax.ShapeDtypeStruct((NC, NP, H), F32),
        mesh=_sc_mesh(),
        scratch_types=[
            pltpu.VMEM((128,), jnp.int32),
            pltpu.VMEM((128,), jnp.int32),
            pltpu.VMEM((128,), jnp.int32),
            pltpu.VMEM((128, H), F32),
            pltpu.VMEM((128, H), F32),
            pltpu.SemaphoreType.DMA,
            pltpu.SemaphoreType.DMA,
            pltpu.VMEM_SHARED((NP, H), F32),
        ],
    )


def _seg_rows(g, src2d, dst2d, zrows):
    return _seg_rows_kernel()(g, src2d, dst2d, zrows)


def _gather_pair_body(g1_hbm, g2_hbm, src_hbm, dst_hbm, a_hbm, b_hbm, idx_s, idx_d, rows, rows2, sem):
    cid = lax.axis_index("c")
    sid = lax.axis_index("s")
    base = (cid * NS + sid) * RPT

    def chunk(j, carry):
        r = base + j
        pltpu.sync_copy(src_hbm.at[r], idx_s)
        pltpu.sync_copy(dst_hbm.at[r], idx_d)
        pltpu.async_copy(g1_hbm.at[idx_s], rows, sem).wait()
        pltpu.sync_copy(rows, a_hbm.at[r])
        pltpu.async_copy(g2_hbm.at[idx_d], rows2, sem).wait()
        pltpu.sync_copy(rows2, b_hbm.at[r])
        return carry

    lax.fori_loop(0, RPT, chunk, 0)


@functools.cache
def _gather_pair_kernel():
    return pl.kernel(
        _gather_pair_body,
        out_type=(
            jax.ShapeDtypeStruct((RP, 128, H), F32),
            jax.ShapeDtypeStruct((RP, 128, H), F32),
        ),
        mesh=_sc_mesh(),
        scratch_types=[
            pltpu.VMEM((128,), jnp.int32),
            pltpu.VMEM((128,), jnp.int32),
            pltpu.VMEM((128, H), F32),
            pltpu.VMEM((128, H), F32),
            pltpu.SemaphoreType.DMA,
        ],
    )


def _gather_pair(G1, G2, src2d, dst2d):
    return _gather_pair_kernel()(G1, G2, src2d, dst2d)


def _seg_masked_body(g_hbm, src_hbm, dst_hbm, seb_hbm, zeros_hbm, out_hbm,
                     idx_s, idx_d, seb_v, rows, acc, sem):
    cid = lax.axis_index("c")
    sid = lax.axis_index("s")
    pltpu.sync_copy(zeros_hbm.at[pl.ds(sid * NPT, NPT)], acc.at[pl.ds(sid * NPT, NPT)])
    plsc.subcore_barrier()
    base = (cid * NS + sid) * RPT

    def chunk(j, carry):
        r = base + j
        pltpu.sync_copy(src_hbm.at[r], idx_s)
        pltpu.sync_copy(dst_hbm.at[r], idx_d)
        pltpu.sync_copy(seb_hbm.at[pl.ds(r * 128, 128)], seb_v)
        pltpu.async_copy(g_hbm.at[idx_s], rows, sem).wait()

        def ebody(e, carry2):
            mk = seb_v[e]
            for k in range(8):
                sl = pl.ds(16 * k, 16)
                rows[e, sl] = rows[e, sl] * mk
            return carry2

        lax.fori_loop(0, 128, ebody, 0)
        pltpu.sync_copy(rows, acc.at[idx_d], add=True)
        return carry

    lax.fori_loop(0, RPT, chunk, 0)
    plsc.subcore_barrier()
    pltpu.sync_copy(acc.at[pl.ds(sid * NPT, NPT)], out_hbm.at[cid, pl.ds(sid * NPT, NPT)])


@functools.cache
def _seg_masked_kernel():
    return pl.kernel(
        _seg_masked_body,
        out_type=jax.ShapeDtypeStruct((NC, NP, H), F32),
        mesh=_sc_mesh(),
        scratch_types=[
            pltpu.VMEM((128,), jnp.int32),
            pltpu.VMEM((128,), jnp.int32),
            pltpu.VMEM((128, L), F32),
            pltpu.VMEM((128, H), F32),
            pltpu.VMEM_SHARED((NP, H), F32),
            pltpu.SemaphoreType.DMA,
        ],
    )


def _seg_masked(g, src2d, dst2d, seb, zrows):
    return _seg_masked_kernel()(g, src2d, dst2d, seb, zrows)


_NBLK = NP // BN
_EBLK = EP // BN


def _mlp_body(x_ref, w1_ref, b1_ref, w2_ref, b2_ref, o_ref):
    h = jnp.maximum(jnp.dot(x_ref[...], w1_ref[...], preferred_element_type=F32) + b1_ref[...], 0.0)
    o_ref[...] = jnp.maximum(jnp.dot(h, w2_ref[...], preferred_element_type=F32) + b2_ref[...], 0.0)


def _tc_mlp(x, w1, b1, w2, b2):
    return pl.pallas_call(
        _mlp_body,
        grid=(_NBLK,),
        in_specs=[
            pl.BlockSpec((BN, H), lambda i: (i, 0)),
            pl.BlockSpec((H, H), lambda i: (0, 0)),
            pl.BlockSpec((1, H), lambda i: (0, 0)),
            pl.BlockSpec((H, H), lambda i: (0, 0)),
            pl.BlockSpec((1, H), lambda i: (0, 0)),
        ],
        out_specs=pl.BlockSpec((BN, H), lambda i: (i, 0)),
        out_shape=jax.ShapeDtypeStruct((NP, H), F32),
    )(x, w1, b1, w2, b2)


def _scale_body(h_ref, degp_ref, g_ref, dinv_ref):
    deg = degp_ref[0, :] + degp_ref[1, :]
    dinv = lax.rsqrt(jnp.maximum(deg, 1.0))
    dinvf = jnp.broadcast_to(dinv[:, None], (BN, H))
    dinv_ref[...] = dinvf
    g_ref[...] = h_ref[...] * dinvf


def _tc_scale(h, degp):
    return pl.pallas_call(
        _scale_body,
        grid=(_NBLK,),
        in_specs=[
            pl.BlockSpec((BN, H), lambda i: (i, 0)),
            pl.BlockSpec((2, BN), lambda i: (0, i)),
        ],
        out_specs=[
            pl.BlockSpec((BN, H), lambda i: (i, 0)),
            pl.BlockSpec((BN, H), lambda i: (i, 0)),
        ],
        out_shape=[
            jax.ShapeDtypeStruct((NP, H), F32),
            jax.ShapeDtypeStruct((NP, H), F32),
        ],
    )(h, degp)


def _fstep_body(base_ref, p_ref, dinv_ref, f_ref, g_ref):
    ps = p_ref[0] + p_ref[1]
    dinv = dinv_ref[...]
    f = base_ref[...] - ps * dinv
    f_ref[...] = f
    g_ref[...] = f * dinv


def _tc_fstep(base, p, dinvf):
    return pl.pallas_call(
        _fstep_body,
        grid=(_NBLK,),
        in_specs=[
            pl.BlockSpec((BN, H), lambda i: (i, 0)),
            pl.BlockSpec((2, BN, H), lambda i: (0, i, 0)),
            pl.BlockSpec((BN, H), lambda i: (i, 0)),
        ],
        out_specs=[
            pl.BlockSpec((BN, H), lambda i: (i, 0)),
            pl.BlockSpec((BN, H), lambda i: (i, 0)),
        ],
        out_shape=[
            jax.ShapeDtypeStruct((NP, H), F32),
            jax.ShapeDtypeStruct((NP, H), F32),
        ],
    )(base, p, dinvf)


def _big_body(h_ref, f1_ref, p_ref, dinv_ref, b0_ref, b1_ref, b2_ref, bias_ref,
              wls_ref, wld_ref, wlmu_ref, bl1_ref,
              f2_ref, g1_ref, g2_ref, c_ref):
    i = pl.program_id(0)
    dinv = dinv_ref[...]
    f2 = f1_ref[...] - (p_ref[0] + p_ref[1]) * dinv
    f2_ref[...] = f2
    h1 = (jnp.dot(h_ref[...], b0_ref[...], preferred_element_type=F32)
          + jnp.dot(f1_ref[...], b1_ref[...], preferred_element_type=F32)
          + jnp.dot(f2, b2_ref[...], preferred_element_type=F32)
          + bias_ref[...])
    g1_ref[...] = jnp.dot(h1, wls_ref[...], preferred_element_type=F32)
    g2_ref[...] = jnp.dot(h1, wld_ref[...], preferred_element_type=F32)
    rowid = i * BN + lax.broadcasted_iota(jnp.int32, (BN, 1), 0)
    h1m = jnp.where(rowid < N, h1, 0.0)
    part = jnp.sum(h1m.reshape(BN // 8, 8, H), axis=0)

    @pl.when(i == 0)
    def _():
        c_ref[...] = jnp.zeros((8, H), F32)

    acc = c_ref[...] + part
    c_ref[...] = acc

    @pl.when(i == _NBLK - 1)
    def _():
        mu = jnp.sum(acc, axis=0, keepdims=True) / float(N)
        cvec = bl1_ref[...] - jnp.dot(mu, wlmu_ref[...], preferred_element_type=F32)
        c_ref[...] = jnp.broadcast_to(cvec, (8, H))


def _tc_big(h, f1, p2, dinvf, b0, b1, b2, bias3, wls, wld, wlmu, bl1):
    full = lambda shape: pl.BlockSpec(shape, lambda i: tuple(0 for _ in shape))
    return pl.pallas_call(
        _big_body,
        grid=(_NBLK,),
        in_specs=[
            pl.BlockSpec((BN, H), lambda i: (i, 0)),
            pl.BlockSpec((BN, H), lambda i: (i, 0)),
            pl.BlockSpec((2, BN, H), lambda i: (0, i, 0)),
            pl.BlockSpec((BN, H), lambda i: (i, 0)),
            full((H, H)), full((H, H)), full((H, H)), full((1, H)),
            full((H, H)), full((H, H)), full((H, H)), full((1, H)),
        ],
        out_specs=[
            pl.BlockSpec((BN, H), lambda i: (i, 0)),
            pl.BlockSpec((BN, H), lambda i: (i, 0)),
            pl.BlockSpec((BN, H), lambda i: (i, 0)),
            pl.BlockSpec((8, H), lambda i: (0, 0)),
        ],
        out_shape=[
            jax.ShapeDtypeStruct((NP, H), F32),
            jax.ShapeDtypeStruct((NP, H), F32),
            jax.ShapeDtypeStruct((NP, H), F32),
            jax.ShapeDtypeStruct((8, H), F32),
        ],
    )(h, f1, p2, dinvf, b0, b1, b2, bias3, wls, wld, wlmu, bl1)


BE = 1024


def _edge_body(a_ref, b_ref, c_ref, leps_ref, wl2_ref, bl2_ref, m2_ref, seb_ref):
    i = pl.program_id(0)
    cvec = c_ref[0:1, :]
    epre = a_ref[...] + b_ref[...] + cvec
    ef = jnp.where(epre > 0, epre, 0.01 * epre)
    s = jnp.sum(ef * wl2_ref[...], axis=1) + bl2_ref[0, 0]
    ef2 = jax.nn.sigmoid(s)
    leps = leps_ref[...].reshape(BE)
    m2 = jax.nn.sigmoid((leps + ef2) * 10.0)
    eid = i * BE + lax.iota(jnp.int32, BE)
    m2 = jnp.where(eid < E, m2, 0.0)
    m2_ref[...] = m2.reshape(BE // 128, 128)
    seb_ref[...] = jnp.broadcast_to(m2[:, None], (BE, L))


def _tc_edge(a, b, cmat, leps2d, wl2row, bl2m):
    return pl.pallas_call(
        _edge_body,
        grid=(EP // BE,),
        in_specs=[
            pl.BlockSpec((BE, H), lambda i: (i, 0)),
            pl.BlockSpec((BE, H), lambda i: (i, 0)),
            pl.BlockSpec((8, H), lambda i: (0, 0)),
            pl.BlockSpec((BE // 128, 128), lambda i: (i, 0)),
            pl.BlockSpec((1, H), lambda i: (0, 0)),
            pl.BlockSpec((1, 1), lambda i: (0, 0)),
        ],
        out_specs=[
            pl.BlockSpec((BE // 128, 128), lambda i: (i, 0)),
            pl.BlockSpec((BE, L), lambda i: (i, 0)),
        ],
        out_shape=[
            jax.ShapeDtypeStruct((RP, 128), F32),
            jax.ShapeDtypeStruct((EP, L), F32),
        ],
    )(a, b, cmat, leps2d, wl2row, bl2m)


def _final_body(h_ref, f1_ref, f2_ref, f1p_ref, q_ref, dinv_ref,
                e0_ref, c1_ref, c2_ref, d1_ref, d2_ref, b31_ref, w4_ref, b4_ref,
                o_ref):
    f2p = f1p_ref[...] - (q_ref[0] + q_ref[1]) * dinv_ref[...]
    h2 = (jnp.dot(h_ref[...], e0_ref[...], preferred_element_type=F32)
          + jnp.dot(f1_ref[...], c1_ref[...], preferred_element_type=F32)
          + jnp.dot(f2_ref[...], c2_ref[...], preferred_element_type=F32)
          + jnp.dot(f1p_ref[...], d1_ref[...], preferred_element_type=F32)
          + jnp.dot(f2p, d2_ref[...], preferred_element_type=F32)
          + b31_ref[...])
    h2 = jnp.maximum(h2, 0.0)
    o_ref[...] = jnp.dot(h2, w4_ref[...], preferred_element_type=F32) + b4_ref[...]


def _tc_final(h, f1, f2, f1p, q2, dinvf, e0, c1, c2, d1, d2, b31, w4p, b4p):
    full = lambda shape: pl.BlockSpec(shape, lambda i: tuple(0 for _ in shape))
    return pl.pallas_call(
        _final_body,
        grid=(_NBLK,),
        in_specs=[
            pl.BlockSpec((BN, H), lambda i: (i, 0)),
            pl.BlockSpec((BN, H), lambda i: (i, 0)),
            pl.BlockSpec((BN, H), lambda i: (i, 0)),
            pl.BlockSpec((BN, H), lambda i: (i, 0)),
            pl.BlockSpec((2, BN, H), lambda i: (0, i, 0)),
            pl.BlockSpec((BN, H), lambda i: (i, 0)),
            full((H, H)), full((H, H)), full((H, H)), full((H, H)), full((H, H)),
            full((1, H)), full((H, H)), full((1, H)),
        ],
        out_specs=pl.BlockSpec((BN, H), lambda i: (i, 0)),
        out_shape=jax.ShapeDtypeStruct((NP, H), F32),
    )(h, f1, f2, f1p, q2, dinvf, e0, c1, c2, d1, d2, b31, w4p, b4p)


def kernel(in_feat, src, dst, W1, b1, W2, b2, W3, b3, W3_1, b3_1, W4, b4, Wl1, bl1, Wl2, bl2):
    th = jnp.asarray(THETAS, F32)
    B = [sum(th[t, k] * W3[t * H:(t + 1) * H] for t in range(3)) for k in range(3)]
    Cc = [sum(th[t, k] * W3_1[t * H:(t + 1) * H] for t in range(3)) for k in range(3)]
    D = [sum(th[t, k] * W3_1[(3 + t) * H:(4 + t) * H] for t in range(3)) for k in range(3)]
    Wla, Wlb, Wlc = Wl1[0:H], Wl1[H:2 * H], Wl1[2 * H:3 * H]
    wls, wld, wlmu = Wla + Wlb, Wlc - Wla, Wlb + Wlc

    x_p = jnp.zeros((NP, H), F32).at[:N].set(in_feat)
    pad_i = jnp.arange(EP - E, dtype=jnp.int32)
    src2d = jnp.concatenate([src, pad_i % N]).reshape(RP, 128)
    dst2d = jnp.concatenate([dst, N + (pad_i % (NP - N))]).reshape(RP, 128)

    bias_u = 0.0001
    eps = (bias_u - (1.0 - bias_u)) * jax.random.uniform(jax.random.key(123), (E,), F32) + (1.0 - bias_u)
    leps = jnp.log(eps) - jnp.log(1.0 - eps)
    leps2d = jnp.concatenate([leps, jnp.zeros((EP - E,), F32)]).reshape(RP, 128)

    zrows = jnp.zeros((NP, H), F32)
    z1d = jnp.zeros((NP,), F32)
    ones2d = jnp.ones((RP, 128), F32)

    b3r = b3.reshape(1, H)
    b31r = b3_1.reshape(1, H)
    bl1r = bl1.reshape(1, H)
    wl2row = Wl2.reshape(1, H)
    bl2m = bl2.reshape(1, 1)
    w4p = jnp.zeros((H, H), F32).at[:, :C].set(W4)
    b4p = jnp.zeros((1, H), F32).at[0, :C].set(b4)

    h = _tc_mlp(x_p, W1, b1.reshape(1, H), W2, b2.reshape(1, H))
    deg1p = _seg_scalar(ones2d, dst2d, z1d)

    g0, dinv1f = _tc_scale(h, deg1p)
    p1 = _seg_rows(g0, src2d, dst2d, zrows)
    f1, g1 = _tc_fstep(h, p1, dinv1f)
    p2 = _seg_rows(g1, src2d, dst2d, zrows)
    f2, G1, G2, cmat = _tc_big(h, f1, p2, dinv1f, B[0], B[1], B[2], b3r,
                               wls, wld, wlmu, bl1r)

    a3, b3d = _gather_pair(G1, G2, src2d, dst2d)
    m2, seb = _tc_edge(a3.reshape(EP, H), b3d.reshape(EP, H), cmat, leps2d, wl2row, bl2m)

    deg2p = _seg_scalar(m2, dst2d, z1d)
    g0p, dinv2f = _tc_scale(h, deg2p)

    q1 = _seg_masked(g0p, src2d, dst2d, seb, zrows)
    f1p, g1p = _tc_fstep(h, q1, dinv2f)
    q2 = _seg_masked(g1p, src2d, dst2d, seb, zrows)

    out = _tc_final(h, f1, f2, f1p, q2, dinv2f,
                    Cc[0] + D[0], Cc[1], Cc[2], D[1], D[2], b31r, w4p, b4p)
    return out[:N, :C]


# speedup vs baseline: 2.6448x; 1.2976x over previous
import functools

import jax
import jax.numpy as jnp
from jax import lax
from jax.experimental import pallas as pl
from jax.experimental.pallas import tpu as pltpu
from jax.experimental.pallas import tpu_sc as plsc

N = 10000
E = 320000
H = 128
C = 2
THETAS = ((3.0, -3.0, 0.75), (0.0, 3.0, -1.5), (0.0, 0.0, 0.75))

NC, NS, L = 2, 16, 16
NP = 10240
RPT = 80
RP = NC * NS * RPT
EP = RP * 128
NPT = NP // NS
BN = 512
F32 = jnp.float32


def _sc_mesh():
    return plsc.VectorSubcoreMesh(core_axis_name="c", subcore_axis_name="s",
                                  num_cores=NC, num_subcores=NS)


def _seg_scalar_body(vals_hbm, dst_hbm, zeros_hbm, out_hbm, idx_d, val_v, acc, sem):
    del sem
    cid = lax.axis_index("c")
    sid = lax.axis_index("s")
    pltpu.sync_copy(zeros_hbm.at[pl.ds(sid * NPT, NPT)], acc.at[pl.ds(sid * NPT, NPT)])
    plsc.subcore_barrier()
    base = (cid * NS + sid) * RPT

    def chunk(j, carry):
        r = base + j
        pltpu.sync_copy(dst_hbm.at[r], idx_d)
        pltpu.sync_copy(vals_hbm.at[r], val_v)
        pltpu.sync_copy(val_v, acc.at[idx_d], add=True)
        return carry

    lax.fori_loop(0, RPT, chunk, 0)
    plsc.subcore_barrier()
    pltpu.sync_copy(acc.at[pl.ds(sid * NPT, NPT)], out_hbm.at[cid, pl.ds(sid * NPT, NPT)])


@functools.cache
def _seg_scalar_kernel():
    return pl.kernel(
        _seg_scalar_body,
        out_type=jax.ShapeDtypeStruct((NC, NP), F32),
        mesh=_sc_mesh(),
        scratch_types=[
            pltpu.VMEM((128,), jnp.int32),
            pltpu.VMEM((128,), F32),
            pltpu.VMEM_SHARED((NP,), F32),
            pltpu.SemaphoreType.DMA,
        ],
    )


def _seg_scalar(vals2d, dst2d, z1d):
    return _seg_scalar_kernel()(vals2d, dst2d, z1d)


def _seg_rows_body(g_hbm, src_hbm, dst_hbm, zeros_hbm, out_hbm,
                   ixs0, ixs1, ixd, rows0, rows1, sg0, sg1, acc):
    cid = lax.axis_index("c")
    sid = lax.axis_index("s")
    pltpu.sync_copy(zeros_hbm.at[pl.ds(sid * NPT, NPT)], acc.at[pl.ds(sid * NPT, NPT)])
    base = (cid * NS + sid) * RPT
    plsc.subcore_barrier()

    pltpu.sync_copy(src_hbm.at[base], ixs0)
    pltpu.async_copy(g_hbm.at[ixs0], rows0, sg0)

    def it(j, carry):
        def step(ixa, ixb, rb, ob, sg, sg_n):
            @pl.when(j + 1 < RPT)
            def _():
                pltpu.sync_copy(src_hbm.at[base + j + 1], ixb)
                pltpu.async_copy(g_hbm.at[ixb], ob, sg_n)

            pltpu.sync_copy(dst_hbm.at[base + j], ixd)
            pltpu.make_async_copy(g_hbm.at[ixa], rb, sg).wait()
            pltpu.sync_copy(rb, acc.at[ixd], add=True)

        @pl.when(j % 2 == 0)
        def _():
            step(ixs0, ixs1, rows0, rows1, sg0, sg1)

        @pl.when(j % 2 == 1)
        def _():
            step(ixs1, ixs0, rows1, rows0, sg1, sg0)

        return carry

    lax.fori_loop(0, RPT, it, 0)
    plsc.subcore_barrier()
    pltpu.sync_copy(acc.at[pl.ds(sid * NPT, NPT)], out_hbm.at[cid, pl.ds(sid * NPT, NPT)])


@functools.cache
def _seg_rows_kernel():
    return pl.kernel(
        _seg_rows_body,
        out_type=jax.ShapeDtypeStruct((NC, NP, H), F32),
        mesh=_sc_mesh(),
        scratch_types=[
            pltpu.VMEM((128,), jnp.int32),
            pltpu.VMEM((128,), jnp.int32),
            pltpu.VMEM((128,), jnp.int32),
            pltpu.VMEM((128, H), F32),
            pltpu.VMEM((128, H), F32),
            pltpu.SemaphoreType.DMA,
            pltpu.SemaphoreType.DMA,
            pltpu.VMEM_SHARED((NP, H), F32),
        ],
    )


def _seg_rows(g, src2d, dst2d, zrows):
    return _seg_rows_kernel()(g, src2d, dst2d, zrows)


def _gather_pair_body(g1_hbm, g2_hbm, src_hbm, dst_hbm, a_hbm, b_hbm,
                      ixs0, ixs1, ixd0, ixd1, ra0, ra1, rb0, rb1,
                      sa0, sa1, sb0, sb1):
    cid = lax.axis_index("c")
    sid = lax.axis_index("s")
    base = (cid * NS + sid) * RPT

    pltpu.sync_copy(src_hbm.at[base], ixs0)
    pltpu.sync_copy(dst_hbm.at[base], ixd0)
    pltpu.async_copy(g1_hbm.at[ixs0], ra0, sa0)
    pltpu.async_copy(g2_hbm.at[ixd0], rb0, sb0)

    def it(j, carry):
        def step(ixsa, ixsb, ixda, ixdb, raa, rab, rba, rbb, sga, sga_n, sgb, sgb_n):
            @pl.when(j + 1 < RPT)
            def _():
                pltpu.sync_copy(src_hbm.at[base + j + 1], ixsb)
                pltpu.sync_copy(dst_hbm.at[base + j + 1], ixdb)
                pltpu.async_copy(g1_hbm.at[ixsb], rab, sga_n)
                pltpu.async_copy(g2_hbm.at[ixdb], rbb, sgb_n)

            pltpu.make_async_copy(g1_hbm.at[ixsa], raa, sga).wait()
            pltpu.sync_copy(raa, a_hbm.at[base + j])
            pltpu.make_async_copy(g2_hbm.at[ixda], rba, sgb).wait()
            pltpu.sync_copy(rba, b_hbm.at[base + j])

        @pl.when(j % 2 == 0)
        def _():
            step(ixs0, ixs1, ixd0, ixd1, ra0, ra1, rb0, rb1, sa0, sa1, sb0, sb1)

        @pl.when(j % 2 == 1)
        def _():
            step(ixs1, ixs0, ixd1, ixd0, ra1, ra0, rb1, rb0, sa1, sa0, sb1, sb0)

        return carry

    lax.fori_loop(0, RPT, it, 0)


@functools.cache
def _gather_pair_kernel():
    return pl.kernel(
        _gather_pair_body,
        out_type=(
            jax.ShapeDtypeStruct((RP, 128, H), F32),
            jax.ShapeDtypeStruct((RP, 128, H), F32),
        ),
        mesh=_sc_mesh(),
        scratch_types=[
            pltpu.VMEM((128,), jnp.int32),
            pltpu.VMEM((128,), jnp.int32),
            pltpu.VMEM((128,), jnp.int32),
            pltpu.VMEM((128,), jnp.int32),
            pltpu.VMEM((128, H), F32),
            pltpu.VMEM((128, H), F32),
            pltpu.VMEM((128, H), F32),
            pltpu.VMEM((128, H), F32),
            pltpu.SemaphoreType.DMA,
            pltpu.SemaphoreType.DMA,
            pltpu.SemaphoreType.DMA,
            pltpu.SemaphoreType.DMA,
        ],
    )


def _gather_pair(G1, G2, src2d, dst2d):
    return _gather_pair_kernel()(G1, G2, src2d, dst2d)


def _seg_masked_body(g_hbm, src_hbm, dst_hbm, m2_hbm, zeros_hbm, out_hbm,
                     ixs2, ixd, m2v, rows0, rows1, sg0, sg1, acc):
    cid = lax.axis_index("c")
    sid = lax.axis_index("s")
    pltpu.sync_copy(zeros_hbm.at[pl.ds(sid * NPT, NPT)], acc.at[pl.ds(sid * NPT, NPT)])
    base = (cid * NS + sid) * RPT
    plsc.subcore_barrier()

    pltpu.sync_copy(src_hbm.at[base], ixs2.at[0])
    pltpu.async_copy(g_hbm.at[ixs2.at[0]], rows0, sg0)

    def it(j, carry):
        def step(pa, pb, rb, ob, sg, sg_n):
            @pl.when(j + 1 < RPT)
            def _():
                pltpu.sync_copy(src_hbm.at[base + j + 1], ixs2.at[pb])
                pltpu.async_copy(g_hbm.at[ixs2.at[pb]], ob, sg_n)

            pltpu.sync_copy(dst_hbm.at[base + j], ixd)
            pltpu.sync_copy(m2_hbm.at[base + j], m2v)
            pltpu.make_async_copy(g_hbm.at[ixs2.at[pa]], rb, sg).wait()

            def gbody(g, carry2):
                mv = m2v[pl.ds(g * L, L)]
                for lane in range(L):
                    e = g * L + lane
                    mk = mv[lane]
                    for k in range(8):
                        sl = pl.ds(16 * k, 16)
                        rb[e, sl] = rb[e, sl] * mk
                return carry2

            lax.fori_loop(0, 8, gbody, 0)
            pltpu.sync_copy(rb, acc.at[ixd], add=True)

        @pl.when(j % 2 == 0)
        def _():
            step(0, 1, rows0, rows1, sg0, sg1)

        @pl.when(j % 2 == 1)
        def _():
            step(1, 0, rows1, rows0, sg1, sg0)

        return carry

    lax.fori_loop(0, RPT, it, 0)
    plsc.subcore_barrier()
    pltpu.sync_copy(acc.at[pl.ds(sid * NPT, NPT)], out_hbm.at[cid, pl.ds(sid * NPT, NPT)])


@functools.cache
def _seg_masked_kernel():
    return pl.kernel(
        _seg_masked_body,
        out_type=jax.ShapeDtypeStruct((NC, NP, H), F32),
        mesh=_sc_mesh(),
        scratch_types=[
            pltpu.VMEM((2, 128), jnp.int32),
            pltpu.VMEM((128,), jnp.int32),
            pltpu.VMEM((128,), F32),
            pltpu.VMEM((128, H), F32),
            pltpu.VMEM((128, H), F32),
            pltpu.SemaphoreType.DMA,
            pltpu.SemaphoreType.DMA,
            pltpu.VMEM_SHARED((NP, H), F32),
        ],
    )


def _seg_masked(g, src2d, dst2d, m2, zrows):
    return _seg_masked_kernel()(g, src2d, dst2d, m2, zrows)


_NBLK = NP // BN
_EBLK = EP // BN


def _mlp_body(x_ref, w1_ref, b1_ref, w2_ref, b2_ref, o_ref):
    h = jnp.maximum(jnp.dot(x_ref[...], w1_ref[...], preferred_element_type=F32) + b1_ref[...], 0.0)
    o_ref[...] = jnp.maximum(jnp.dot(h, w2_ref[...], preferred_element_type=F32) + b2_ref[...], 0.0)


def _tc_mlp(x, w1, b1, w2, b2):
    return pl.pallas_call(
        _mlp_body,
        grid=(_NBLK,),
        in_specs=[
            pl.BlockSpec((BN, H), lambda i: (i, 0)),
            pl.BlockSpec((H, H), lambda i: (0, 0)),
            pl.BlockSpec((1, H), lambda i: (0, 0)),
            pl.BlockSpec((H, H), lambda i: (0, 0)),
            pl.BlockSpec((1, H), lambda i: (0, 0)),
        ],
        out_specs=pl.BlockSpec((BN, H), lambda i: (i, 0)),
        out_shape=jax.ShapeDtypeStruct((NP, H), F32),
    )(x, w1, b1, w2, b2)


def _scale_body(h_ref, degp_ref, g_ref, dinv_ref):
    deg = degp_ref[0, :] + degp_ref[1, :]
    dinv = lax.rsqrt(jnp.maximum(deg, 1.0))
    dinvf = jnp.broadcast_to(dinv[:, None], (BN, H))
    dinv_ref[...] = dinvf
    g_ref[...] = h_ref[...] * dinvf


def _tc_scale(h, degp):
    return pl.pallas_call(
        _scale_body,
        grid=(_NBLK,),
        in_specs=[
            pl.BlockSpec((BN, H), lambda i: (i, 0)),
            pl.BlockSpec((2, BN), lambda i: (0, i)),
        ],
        out_specs=[
            pl.BlockSpec((BN, H), lambda i: (i, 0)),
            pl.BlockSpec((BN, H), lambda i: (i, 0)),
        ],
        out_shape=[
            jax.ShapeDtypeStruct((NP, H), F32),
            jax.ShapeDtypeStruct((NP, H), F32),
        ],
    )(h, degp)


def _fstep_body(base_ref, p_ref, dinv_ref, f_ref, g_ref):
    ps = p_ref[0] + p_ref[1]
    dinv = dinv_ref[...]
    f = base_ref[...] - ps * dinv
    f_ref[...] = f
    g_ref[...] = f * dinv


def _tc_fstep(base, p, dinvf):
    return pl.pallas_call(
        _fstep_body,
        grid=(_NBLK,),
        in_specs=[
            pl.BlockSpec((BN, H), lambda i: (i, 0)),
            pl.BlockSpec((2, BN, H), lambda i: (0, i, 0)),
            pl.BlockSpec((BN, H), lambda i: (i, 0)),
        ],
        out_specs=[
            pl.BlockSpec((BN, H), lambda i: (i, 0)),
            pl.BlockSpec((BN, H), lambda i: (i, 0)),
        ],
        out_shape=[
            jax.ShapeDtypeStruct((NP, H), F32),
            jax.ShapeDtypeStruct((NP, H), F32),
        ],
    )(base, p, dinvf)


def _big_body(h_ref, f1_ref, p_ref, dinv_ref, b0_ref, b1_ref, b2_ref, bias_ref,
              wls_ref, wld_ref, wlmu_ref, bl1_ref,
              f2_ref, g1_ref, g2_ref, c_ref):
    i = pl.program_id(0)
    dinv = dinv_ref[...]
    f2 = f1_ref[...] - (p_ref[0] + p_ref[1]) * dinv
    f2_ref[...] = f2
    h1 = (jnp.dot(h_ref[...], b0_ref[...], preferred_element_type=F32)
          + jnp.dot(f1_ref[...], b1_ref[...], preferred_element_type=F32)
          + jnp.dot(f2, b2_ref[...], preferred_element_type=F32)
          + bias_ref[...])
    g1_ref[...] = jnp.dot(h1, wls_ref[...], preferred_element_type=F32)
    g2_ref[...] = jnp.dot(h1, wld_ref[...], preferred_element_type=F32)
    rowid = i * BN + lax.broadcasted_iota(jnp.int32, (BN, 1), 0)
    h1m = jnp.where(rowid < N, h1, 0.0)
    part = jnp.sum(h1m.reshape(BN // 8, 8, H), axis=0)

    @pl.when(i == 0)
    def _():
        c_ref[...] = jnp.zeros((8, H), F32)

    acc = c_ref[...] + part
    c_ref[...] = acc

    @pl.when(i == _NBLK - 1)
    def _():
        mu = jnp.sum(acc, axis=0, keepdims=True) / float(N)
        cvec = bl1_ref[...] - jnp.dot(mu, wlmu_ref[...], preferred_element_type=F32)
        c_ref[...] = jnp.broadcast_to(cvec, (8, H))


def _tc_big(h, f1, p2, dinvf, b0, b1, b2, bias3, wls, wld, wlmu, bl1):
    full = lambda shape: pl.BlockSpec(shape, lambda i: tuple(0 for _ in shape))
    return pl.pallas_call(
        _big_body,
        grid=(_NBLK,),
        in_specs=[
            pl.BlockSpec((BN, H), lambda i: (i, 0)),
            pl.BlockSpec((BN, H), lambda i: (i, 0)),
            pl.BlockSpec((2, BN, H), lambda i: (0, i, 0)),
            pl.BlockSpec((BN, H), lambda i: (i, 0)),
            full((H, H)), full((H, H)), full((H, H)), full((1, H)),
            full((H, H)), full((H, H)), full((H, H)), full((1, H)),
        ],
        out_specs=[
            pl.BlockSpec((BN, H), lambda i: (i, 0)),
            pl.BlockSpec((BN, H), lambda i: (i, 0)),
            pl.BlockSpec((BN, H), lambda i: (i, 0)),
            pl.BlockSpec((8, H), lambda i: (0, 0)),
        ],
        out_shape=[
            jax.ShapeDtypeStruct((NP, H), F32),
            jax.ShapeDtypeStruct((NP, H), F32),
            jax.ShapeDtypeStruct((NP, H), F32),
            jax.ShapeDtypeStruct((8, H), F32),
        ],
    )(h, f1, p2, dinvf, b0, b1, b2, bias3, wls, wld, wlmu, bl1)


BE = 1024


def _edge_body(a_ref, b_ref, c_ref, leps_ref, wl2_ref, bl2_ref, m2_ref, seb_ref):
    i = pl.program_id(0)
    cvec = c_ref[0:1, :]
    epre = a_ref[...] + b_ref[...] + cvec
    ef = jnp.where(epre > 0, epre, 0.01 * epre)
    s = jnp.sum(ef * wl2_ref[...], axis=1) + bl2_ref[0, 0]
    ef2 = jax.nn.sigmoid(s)
    leps = leps_ref[...].reshape(BE)
    m2 = jax.nn.sigmoid((leps + ef2) * 10.0)
    eid = i * BE + lax.iota(jnp.int32, BE)
    m2 = jnp.where(eid < E, m2, 0.0)
    m2_ref[...] = m2.reshape(BE // 128, 128)
    seb_ref[...] = jnp.broadcast_to(m2[:, None], (BE, L))


def _tc_edge(a, b, cmat, leps2d, wl2row, bl2m):
    return pl.pallas_call(
        _edge_body,
        grid=(EP // BE,),
        in_specs=[
            pl.BlockSpec((BE, H), lambda i: (i, 0)),
            pl.BlockSpec((BE, H), lambda i: (i, 0)),
            pl.BlockSpec((8, H), lambda i: (0, 0)),
            pl.BlockSpec((BE // 128, 128), lambda i: (i, 0)),
            pl.BlockSpec((1, H), lambda i: (0, 0)),
            pl.BlockSpec((1, 1), lambda i: (0, 0)),
        ],
        out_specs=[
            pl.BlockSpec((BE // 128, 128), lambda i: (i, 0)),
            pl.BlockSpec((BE, L), lambda i: (i, 0)),
        ],
        out_shape=[
            jax.ShapeDtypeStruct((RP, 128), F32),
            jax.ShapeDtypeStruct((EP, L), F32),
        ],
    )(a, b, cmat, leps2d, wl2row, bl2m)


def _final_body(h_ref, f1_ref, f2_ref, f1p_ref, q_ref, dinv_ref,
                e0_ref, c1_ref, c2_ref, d1_ref, d2_ref, b31_ref, w4_ref, b4_ref,
                o_ref):
    f2p = f1p_ref[...] - (q_ref[0] + q_ref[1]) * dinv_ref[...]
    h2 = (jnp.dot(h_ref[...], e0_ref[...], preferred_element_type=F32)
          + jnp.dot(f1_ref[...], c1_ref[...], preferred_element_type=F32)
          + jnp.dot(f2_ref[...], c2_ref[...], preferred_element_type=F32)
          + jnp.dot(f1p_ref[...], d1_ref[...], preferred_element_type=F32)
          + jnp.dot(f2p, d2_ref[...], preferred_element_type=F32)
          + b31_ref[...])
    h2 = jnp.maximum(h2, 0.0)
    o_ref[...] = jnp.dot(h2, w4_ref[...], preferred_element_type=F32) + b4_ref[...]


def _tc_final(h, f1, f2, f1p, q2, dinvf, e0, c1, c2, d1, d2, b31, w4p, b4p):
    full = lambda shape: pl.BlockSpec(shape, lambda i: tuple(0 for _ in shape))
    return pl.pallas_call(
        _final_body,
        grid=(_NBLK,),
        in_specs=[
            pl.BlockSpec((BN, H), lambda i: (i, 0)),
            pl.BlockSpec((BN, H), lambda i: (i, 0)),
            pl.BlockSpec((BN, H), lambda i: (i, 0)),
            pl.BlockSpec((BN, H), lambda i: (i, 0)),
            pl.BlockSpec((2, BN, H), lambda i: (0, i, 0)),
            pl.BlockSpec((BN, H), lambda i: (i, 0)),
            full((H, H)), full((H, H)), full((H, H)), full((H, H)), full((H, H)),
            full((1, H)), full((H, H)), full((1, H)),
        ],
        out_specs=pl.BlockSpec((BN, H), lambda i: (i, 0)),
        out_shape=jax.ShapeDtypeStruct((NP, H), F32),
    )(h, f1, f2, f1p, q2, dinvf, e0, c1, c2, d1, d2, b31, w4p, b4p)


def kernel(in_feat, src, dst, W1, b1, W2, b2, W3, b3, W3_1, b3_1, W4, b4, Wl1, bl1, Wl2, bl2):
    th = jnp.asarray(THETAS, F32)
    B = [sum(th[t, k] * W3[t * H:(t + 1) * H] for t in range(3)) for k in range(3)]
    Cc = [sum(th[t, k] * W3_1[t * H:(t + 1) * H] for t in range(3)) for k in range(3)]
    D = [sum(th[t, k] * W3_1[(3 + t) * H:(4 + t) * H] for t in range(3)) for k in range(3)]
    Wla, Wlb, Wlc = Wl1[0:H], Wl1[H:2 * H], Wl1[2 * H:3 * H]
    wls, wld, wlmu = Wla + Wlb, Wlc - Wla, Wlb + Wlc

    x_p = jnp.zeros((NP, H), F32).at[:N].set(in_feat)
    pad_i = jnp.arange(EP - E, dtype=jnp.int32)
    src2d = jnp.concatenate([src, pad_i % N]).reshape(RP, 128)
    dst2d = jnp.concatenate([dst, N + (pad_i % (NP - N))]).reshape(RP, 128)

    bias_u = 0.0001
    eps = (bias_u - (1.0 - bias_u)) * jax.random.uniform(jax.random.key(123), (E,), F32) + (1.0 - bias_u)
    leps = jnp.log(eps) - jnp.log(1.0 - eps)
    leps2d = jnp.concatenate([leps, jnp.zeros((EP - E,), F32)]).reshape(RP, 128)

    zrows = jnp.zeros((NP, H), F32)
    z1d = jnp.zeros((NP,), F32)
    ones2d = jnp.ones((RP, 128), F32)

    b3r = b3.reshape(1, H)
    b31r = b3_1.reshape(1, H)
    bl1r = bl1.reshape(1, H)
    wl2row = Wl2.reshape(1, H)
    bl2m = bl2.reshape(1, 1)
    w4p = jnp.zeros((H, H), F32).at[:, :C].set(W4)
    b4p = jnp.zeros((1, H), F32).at[0, :C].set(b4)

    h = _tc_mlp(x_p, W1, b1.reshape(1, H), W2, b2.reshape(1, H))
    deg1p = _seg_scalar(ones2d, dst2d, z1d)

    g0, dinv1f = _tc_scale(h, deg1p)
    p1 = _seg_rows(g0, src2d, dst2d, zrows)
    f1, g1 = _tc_fstep(h, p1, dinv1f)
    p2 = _seg_rows(g1, src2d, dst2d, zrows)
    f2, G1, G2, cmat = _tc_big(h, f1, p2, dinv1f, B[0], B[1], B[2], b3r,
                               wls, wld, wlmu, bl1r)

    a3, b3d = _gather_pair(G1, G2, src2d, dst2d)
    m2, seb = _tc_edge(a3.reshape(EP, H), b3d.reshape(EP, H), cmat, leps2d, wl2row, bl2m)

    deg2p = _seg_scalar(m2, dst2d, z1d)
    g0p, dinv2f = _tc_scale(h, deg2p)

    q1 = _seg_masked(g0p, src2d, dst2d, m2, zrows)
    f1p, g1p = _tc_fstep(h, q1, dinv2f)
    q2 = _seg_masked(g1p, src2d, dst2d, m2, zrows)

    out = _tc_final(h, f1, f2, f1p, q2, dinv2f,
                    Cc[0] + D[0], Cc[1], Cc[2], D[1], D[2], b31r, w4p, b4p)
    return out[:N, :C]


# speedup vs baseline: 2.6778x; 1.0124x over previous
import functools

import jax
import jax.numpy as jnp
from jax import lax
from jax.experimental import pallas as pl
from jax.experimental.pallas import tpu as pltpu
from jax.experimental.pallas import tpu_sc as plsc

N = 10000
E = 320000
H = 128
C = 2
THETAS = ((3.0, -3.0, 0.75), (0.0, 3.0, -1.5), (0.0, 0.0, 0.75))

NC, NS, L = 2, 16, 16
NP = 10240
RPT = 80
RP = NC * NS * RPT
EP = RP * 128
NPT = NP // NS
BN = 512
F32 = jnp.float32


def _sc_mesh():
    return plsc.VectorSubcoreMesh(core_axis_name="c", subcore_axis_name="s",
                                  num_cores=NC, num_subcores=NS)


def _seg_scalar_body(vals_hbm, dst_hbm, zeros_hbm, out_hbm, idx_d, val_v, acc, sem):
    del sem
    cid = lax.axis_index("c")
    sid = lax.axis_index("s")
    pltpu.sync_copy(zeros_hbm.at[pl.ds(sid * NPT, NPT)], acc.at[pl.ds(sid * NPT, NPT)])
    plsc.subcore_barrier()
    base = (cid * NS + sid) * RPT

    def chunk(j, carry):
        r = base + j
        pltpu.sync_copy(dst_hbm.at[r], idx_d)
        pltpu.sync_copy(vals_hbm.at[r], val_v)
        pltpu.sync_copy(val_v, acc.at[idx_d], add=True)
        return carry

    lax.fori_loop(0, RPT, chunk, 0)
    plsc.subcore_barrier()
    pltpu.sync_copy(acc.at[pl.ds(sid * NPT, NPT)], out_hbm.at[cid, pl.ds(sid * NPT, NPT)])


@functools.cache
def _seg_scalar_kernel():
    return pl.kernel(
        _seg_scalar_body,
        out_type=jax.ShapeDtypeStruct((NC, NP), F32),
        mesh=_sc_mesh(),
        scratch_types=[
            pltpu.VMEM((128,), jnp.int32),
            pltpu.VMEM((128,), F32),
            pltpu.VMEM_SHARED((NP,), F32),
            pltpu.SemaphoreType.DMA,
        ],
    )


def _seg_scalar(vals2d, dst2d, z1d):
    return _seg_scalar_kernel()(vals2d, dst2d, z1d)


def _seg_rows_body(g_hbm, src_hbm, dst_hbm, zeros_hbm, out_hbm,
                   ixs0, ixs1, ixd, rows0, rows1, sg0, sg1, acc):
    cid = lax.axis_index("c")
    sid = lax.axis_index("s")
    pltpu.sync_copy(zeros_hbm.at[pl.ds(sid * NPT, NPT)], acc.at[pl.ds(sid * NPT, NPT)])
    base = (cid * NS + sid) * RPT
    plsc.subcore_barrier()

    pltpu.sync_copy(src_hbm.at[base], ixs0)
    pltpu.async_copy(g_hbm.at[ixs0], rows0, sg0)

    def it(j, carry):
        def step(ixa, ixb, rb, ob, sg, sg_n):
            @pl.when(j + 1 < RPT)
            def _():
                pltpu.sync_copy(src_hbm.at[base + j + 1], ixb)
                pltpu.async_copy(g_hbm.at[ixb], ob, sg_n)

            pltpu.sync_copy(dst_hbm.at[base + j], ixd)
            pltpu.make_async_copy(g_hbm.at[ixa], rb, sg).wait()
            pltpu.sync_copy(rb, acc.at[ixd], add=True)

        @pl.when(j % 2 == 0)
        def _():
            step(ixs0, ixs1, rows0, rows1, sg0, sg1)

        @pl.when(j % 2 == 1)
        def _():
            step(ixs1, ixs0, rows1, rows0, sg1, sg0)

        return carry

    lax.fori_loop(0, RPT, it, 0)
    plsc.subcore_barrier()
    pltpu.sync_copy(acc.at[pl.ds(sid * NPT, NPT)], out_hbm.at[cid, pl.ds(sid * NPT, NPT)])


@functools.cache
def _seg_rows_kernel():
    return pl.kernel(
        _seg_rows_body,
        out_type=jax.ShapeDtypeStruct((NC, NP, H), F32),
        mesh=_sc_mesh(),
        scratch_types=[
            pltpu.VMEM((128,), jnp.int32),
            pltpu.VMEM((128,), jnp.int32),
            pltpu.VMEM((128,), jnp.int32),
            pltpu.VMEM((128, H), F32),
            pltpu.VMEM((128, H), F32),
            pltpu.SemaphoreType.DMA,
            pltpu.SemaphoreType.DMA,
            pltpu.VMEM_SHARED((NP, H), F32),
        ],
    )


def _seg_rows(g, src2d, dst2d, zrows):
    return _seg_rows_kernel()(g, src2d, dst2d, zrows)


def _gather_pair_body(g1_hbm, g2_hbm, src_hbm, dst_hbm, a_hbm, b_hbm,
                      ixs0, ixs1, ixd0, ixd1, ra0, ra1, rb0, rb1,
                      sa0, sa1, sb0, sb1):
    cid = lax.axis_index("c")
    sid = lax.axis_index("s")
    base = (cid * NS + sid) * RPT

    pltpu.sync_copy(src_hbm.at[base], ixs0)
    pltpu.sync_copy(dst_hbm.at[base], ixd0)
    pltpu.async_copy(g1_hbm.at[ixs0], ra0, sa0)
    pltpu.async_copy(g2_hbm.at[ixd0], rb0, sb0)

    def it(j, carry):
        def step(ixsa, ixsb, ixda, ixdb, raa, rab, rba, rbb, sga, sga_n, sgb, sgb_n):
            @pl.when(j + 1 < RPT)
            def _():
                pltpu.sync_copy(src_hbm.at[base + j + 1], ixsb)
                pltpu.sync_copy(dst_hbm.at[base + j + 1], ixdb)
                pltpu.async_copy(g1_hbm.at[ixsb], rab, sga_n)
                pltpu.async_copy(g2_hbm.at[ixdb], rbb, sgb_n)

            pltpu.make_async_copy(g1_hbm.at[ixsa], raa, sga).wait()
            pltpu.sync_copy(raa, a_hbm.at[base + j])
            pltpu.make_async_copy(g2_hbm.at[ixda], rba, sgb).wait()
            pltpu.sync_copy(rba, b_hbm.at[base + j])

        @pl.when(j % 2 == 0)
        def _():
            step(ixs0, ixs1, ixd0, ixd1, ra0, ra1, rb0, rb1, sa0, sa1, sb0, sb1)

        @pl.when(j % 2 == 1)
        def _():
            step(ixs1, ixs0, ixd1, ixd0, ra1, ra0, rb1, rb0, sa1, sa0, sb1, sb0)

        return carry

    lax.fori_loop(0, RPT, it, 0)


@functools.cache
def _gather_pair_kernel():
    return pl.kernel(
        _gather_pair_body,
        out_type=(
            jax.ShapeDtypeStruct((RP, 128, H), F32),
            jax.ShapeDtypeStruct((RP, 128, H), F32),
        ),
        mesh=_sc_mesh(),
        scratch_types=[
            pltpu.VMEM((128,), jnp.int32),
            pltpu.VMEM((128,), jnp.int32),
            pltpu.VMEM((128,), jnp.int32),
            pltpu.VMEM((128,), jnp.int32),
            pltpu.VMEM((128, H), F32),
            pltpu.VMEM((128, H), F32),
            pltpu.VMEM((128, H), F32),
            pltpu.VMEM((128, H), F32),
            pltpu.SemaphoreType.DMA,
            pltpu.SemaphoreType.DMA,
            pltpu.SemaphoreType.DMA,
            pltpu.SemaphoreType.DMA,
        ],
    )


def _gather_pair(G1, G2, src2d, dst2d):
    return _gather_pair_kernel()(G1, G2, src2d, dst2d)


def _seg_masked_body(g_hbm, src_hbm, dst_hbm, m2_hbm, zeros_hbm, out_hbm,
                     ixs2, ixd, m2v, rows0, rows1, sg0, sg1, acc):
    cid = lax.axis_index("c")
    sid = lax.axis_index("s")
    pltpu.sync_copy(zeros_hbm.at[pl.ds(sid * NPT, NPT)], acc.at[pl.ds(sid * NPT, NPT)])
    base = (cid * NS + sid) * RPT
    plsc.subcore_barrier()

    pltpu.sync_copy(src_hbm.at[base], ixs2.at[0])
    pltpu.async_copy(g_hbm.at[ixs2.at[0]], rows0, sg0)

    def it(j, carry):
        def step(pa, pb, rb, ob, sg, sg_n):
            @pl.when(j + 1 < RPT)
            def _():
                pltpu.sync_copy(src_hbm.at[base + j + 1], ixs2.at[pb])
                pltpu.async_copy(g_hbm.at[ixs2.at[pb]], ob, sg_n)

            pltpu.sync_copy(dst_hbm.at[base + j], ixd)
            pltpu.sync_copy(m2_hbm.at[base + j], m2v)
            pltpu.make_async_copy(g_hbm.at[ixs2.at[pa]], rb, sg).wait()

            def gbody(g, carry2):
                mv = m2v[pl.ds(g * L, L)]
                for lane in range(L):
                    e = g * L + lane
                    mk = mv[lane]
                    for k in range(8):
                        sl = pl.ds(16 * k, 16)
                        rb[e, sl] = rb[e, sl] * mk
                return carry2

            lax.fori_loop(0, 8, gbody, 0)
            pltpu.sync_copy(rb, acc.at[ixd], add=True)

        @pl.when(j % 2 == 0)
        def _():
            step(0, 1, rows0, rows1, sg0, sg1)

        @pl.when(j % 2 == 1)
        def _():
            step(1, 0, rows1, rows0, sg1, sg0)

        return carry

    lax.fori_loop(0, RPT, it, 0)
    plsc.subcore_barrier()
    pltpu.sync_copy(acc.at[pl.ds(sid * NPT, NPT)], out_hbm.at[cid, pl.ds(sid * NPT, NPT)])


@functools.cache
def _seg_masked_kernel():
    return pl.kernel(
        _seg_masked_body,
        out_type=jax.ShapeDtypeStruct((NC, NP, H), F32),
        mesh=_sc_mesh(),
        scratch_types=[
            pltpu.VMEM((2, 128), jnp.int32),
            pltpu.VMEM((128,), jnp.int32),
            pltpu.VMEM((128,), F32),
            pltpu.VMEM((128, H), F32),
            pltpu.VMEM((128, H), F32),
            pltpu.SemaphoreType.DMA,
            pltpu.SemaphoreType.DMA,
            pltpu.VMEM_SHARED((NP, H), F32),
        ],
    )


def _seg_masked(g, src2d, dst2d, m2, zrows):
    return _seg_masked_kernel()(g, src2d, dst2d, m2, zrows)


_NBLK = NP // BN
_EBLK = EP // BN


def _mlp_body(x_ref, w1_ref, b1_ref, w2_ref, b2_ref, o_ref):
    h = jnp.maximum(jnp.dot(x_ref[...], w1_ref[...], preferred_element_type=F32) + b1_ref[...], 0.0)
    o_ref[...] = jnp.maximum(jnp.dot(h, w2_ref[...], preferred_element_type=F32) + b2_ref[...], 0.0)


def _tc_mlp(x, w1, b1, w2, b2):
    return pl.pallas_call(
        _mlp_body,
        grid=(_NBLK,),
        in_specs=[
            pl.BlockSpec((BN, H), lambda i: (i, 0)),
            pl.BlockSpec((H, H), lambda i: (0, 0)),
            pl.BlockSpec((1, H), lambda i: (0, 0)),
            pl.BlockSpec((H, H), lambda i: (0, 0)),
            pl.BlockSpec((1, H), lambda i: (0, 0)),
        ],
        out_specs=pl.BlockSpec((BN, H), lambda i: (i, 0)),
        out_shape=jax.ShapeDtypeStruct((NP, H), F32),
    )(x, w1, b1, w2, b2)


def _scale_body(h_ref, degp_ref, g_ref, dinv_ref):
    deg = degp_ref[0, :] + degp_ref[1, :]
    dinv = jnp.power(jnp.maximum(deg, 1.0), -0.5)
    dinvf = jnp.broadcast_to(dinv[:, None], (BN, H))
    dinv_ref[...] = dinvf
    g_ref[...] = h_ref[...] * dinvf


def _tc_scale(h, degp):
    return pl.pallas_call(
        _scale_body,
        grid=(_NBLK,),
        in_specs=[
            pl.BlockSpec((BN, H), lambda i: (i, 0)),
            pl.BlockSpec((2, BN), lambda i: (0, i)),
        ],
        out_specs=[
            pl.BlockSpec((BN, H), lambda i: (i, 0)),
            pl.BlockSpec((BN, H), lambda i: (i, 0)),
        ],
        out_shape=[
            jax.ShapeDtypeStruct((NP, H), F32),
            jax.ShapeDtypeStruct((NP, H), F32),
        ],
    )(h, degp)


def _fstep_body(base_ref, p_ref, dinv_ref, f_ref, g_ref):
    ps = p_ref[0] + p_ref[1]
    dinv = dinv_ref[...]
    f = base_ref[...] - ps * dinv
    f_ref[...] = f
    g_ref[...] = f * dinv


def _tc_fstep(base, p, dinvf):
    return pl.pallas_call(
        _fstep_body,
        grid=(_NBLK,),
        in_specs=[
            pl.BlockSpec((BN, H), lambda i: (i, 0)),
            pl.BlockSpec((2, BN, H), lambda i: (0, i, 0)),
            pl.BlockSpec((BN, H), lambda i: (i, 0)),
        ],
        out_specs=[
            pl.BlockSpec((BN, H), lambda i: (i, 0)),
            pl.BlockSpec((BN, H), lambda i: (i, 0)),
        ],
        out_shape=[
            jax.ShapeDtypeStruct((NP, H), F32),
            jax.ShapeDtypeStruct((NP, H), F32),
        ],
    )(base, p, dinvf)


def _big_body(h_ref, f1_ref, p_ref, dinv_ref, w30_ref, w31_ref, w32_ref, bias_ref,
              f2_ref, h1_ref, mu_ref):
    i = pl.program_id(0)
    dinv = dinv_ref[...]
    h = h_ref[...]
    f1 = f1_ref[...]
    f2 = f1 - (p_ref[0] + p_ref[1]) * dinv
    f2_ref[...] = f2
    hs0 = (3.0 * h + (-3.0) * f1) + 0.75 * f2
    hs1 = (0.0 * h + 3.0 * f1) + (-1.5) * f2
    hs2 = (0.0 * h + 0.0 * f1) + 0.75 * f2
    h1 = ((jnp.dot(hs0, w30_ref[...], preferred_element_type=F32)
           + jnp.dot(hs1, w31_ref[...], preferred_element_type=F32))
          + jnp.dot(hs2, w32_ref[...], preferred_element_type=F32)) + bias_ref[...]
    h1_ref[...] = h1
    rowid = i * BN + lax.broadcasted_iota(jnp.int32, (BN, 1), 0)
    h1m = jnp.where(rowid < N, h1, 0.0)
    part = jnp.sum(h1m.reshape(BN // 8, 8, H), axis=0)

    @pl.when(i == 0)
    def _():
        mu_ref[...] = jnp.zeros((8, H), F32)

    mu_ref[...] += part


def _tc_big(h, f1, p2, dinvf, w30, w31, w32, bias3):
    full = lambda shape: pl.BlockSpec(shape, lambda i: tuple(0 for _ in shape))
    return pl.pallas_call(
        _big_body,
        grid=(_NBLK,),
        in_specs=[
            pl.BlockSpec((BN, H), lambda i: (i, 0)),
            pl.BlockSpec((BN, H), lambda i: (i, 0)),
            pl.BlockSpec((2, BN, H), lambda i: (0, i, 0)),
            pl.BlockSpec((BN, H), lambda i: (i, 0)),
            full((H, H)), full((H, H)), full((H, H)), full((1, H)),
        ],
        out_specs=[
            pl.BlockSpec((BN, H), lambda i: (i, 0)),
            pl.BlockSpec((BN, H), lambda i: (i, 0)),
            pl.BlockSpec((8, H), lambda i: (0, 0)),
        ],
        out_shape=[
            jax.ShapeDtypeStruct((NP, H), F32),
            jax.ShapeDtypeStruct((NP, H), F32),
            jax.ShapeDtypeStruct((8, H), F32),
        ],
    )(h, f1, p2, dinvf, w30, w31, w32, bias3)


BE = 1024


def _edge_body(a_ref, b_ref, mu_ref, leps_ref, wla_ref, wlb_ref, wlc_ref,
               bl1_ref, wl2_ref, bl2_ref, m2_ref):
    i = pl.program_id(0)
    mu = jnp.sum(mu_ref[...], axis=0, keepdims=True) / float(N)
    a = a_ref[...]
    b = b_ref[...]
    ef = (((jnp.dot(a - b, wla_ref[...], preferred_element_type=F32)
            + jnp.dot(a - mu, wlb_ref[...], preferred_element_type=F32))
           + jnp.dot(b - mu, wlc_ref[...], preferred_element_type=F32))
          + bl1_ref[...])
    ef = jnp.where(ef > 0, ef, 0.01 * ef)
    s = jnp.sum(ef * wl2_ref[...], axis=1) + bl2_ref[0, 0]
    ef2 = jax.nn.sigmoid(s)
    leps = leps_ref[...].reshape(BE)
    m2 = jax.nn.sigmoid((leps + ef2) / 0.1)
    eid = i * BE + lax.iota(jnp.int32, BE)
    m2 = jnp.where(eid < E, m2, 0.0)
    m2_ref[...] = m2.reshape(BE // 128, 128)


def _tc_edge(a, b, musum, leps2d, wla, wlb, wlc, bl1r, wl2row, bl2m):
    full = lambda shape: pl.BlockSpec(shape, lambda i: tuple(0 for _ in shape))
    return pl.pallas_call(
        _edge_body,
        grid=(EP // BE,),
        in_specs=[
            pl.BlockSpec((BE, H), lambda i: (i, 0)),
            pl.BlockSpec((BE, H), lambda i: (i, 0)),
            full((8, H)),
            pl.BlockSpec((BE // 128, 128), lambda i: (i, 0)),
            full((H, H)), full((H, H)), full((H, H)),
            full((1, H)), full((1, H)), full((1, 1)),
        ],
        out_specs=pl.BlockSpec((BE // 128, 128), lambda i: (i, 0)),
        out_shape=jax.ShapeDtypeStruct((RP, 128), F32),
    )(a, b, musum, leps2d, wla, wlb, wlc, bl1r, wl2row, bl2m)


def _final_body(h_ref, f1_ref, f2_ref, f1p_ref, q_ref, dinv_ref,
                w0_ref, w1_ref, w2_ref, w3_ref, w4w_ref, w5_ref,
                b31_ref, w4_ref, b4_ref, o_ref):
    h = h_ref[...]
    f1 = f1_ref[...]
    f2 = f2_ref[...]
    f1p = f1p_ref[...]
    f2p = f1p - (q_ref[0] + q_ref[1]) * dinv_ref[...]
    hs0 = (3.0 * h + (-3.0) * f1) + 0.75 * f2
    hs1 = (0.0 * h + 3.0 * f1) + (-1.5) * f2
    hs2 = (0.0 * h + 0.0 * f1) + 0.75 * f2
    hp0 = (3.0 * h + (-3.0) * f1p) + 0.75 * f2p
    hp1 = (0.0 * h + 3.0 * f1p) + (-1.5) * f2p
    hp2 = (0.0 * h + 0.0 * f1p) + 0.75 * f2p
    h2 = (((((jnp.dot(hs0, w0_ref[...], preferred_element_type=F32)
              + jnp.dot(hs1, w1_ref[...], preferred_element_type=F32))
             + jnp.dot(hs2, w2_ref[...], preferred_element_type=F32))
            + jnp.dot(hp0, w3_ref[...], preferred_element_type=F32))
           + jnp.dot(hp1, w4w_ref[...], preferred_element_type=F32))
          + jnp.dot(hp2, w5_ref[...], preferred_element_type=F32)) + b31_ref[...]
    h2 = jnp.maximum(h2, 0.0)
    o_ref[...] = jnp.dot(h2, w4_ref[...], preferred_element_type=F32) + b4_ref[...]


def _tc_final(h, f1, f2, f1p, q2, dinvf, ws, b31, w4p, b4p):
    full = lambda shape: pl.BlockSpec(shape, lambda i: tuple(0 for _ in shape))
    return pl.pallas_call(
        _final_body,
        grid=(_NBLK,),
        in_specs=[
            pl.BlockSpec((BN, H), lambda i: (i, 0)),
            pl.BlockSpec((BN, H), lambda i: (i, 0)),
            pl.BlockSpec((BN, H), lambda i: (i, 0)),
            pl.BlockSpec((BN, H), lambda i: (i, 0)),
            pl.BlockSpec((2, BN, H), lambda i: (0, i, 0)),
            pl.BlockSpec((BN, H), lambda i: (i, 0)),
            full((H, H)), full((H, H)), full((H, H)),
            full((H, H)), full((H, H)), full((H, H)),
            full((1, H)), full((H, H)), full((1, H)),
        ],
        out_specs=pl.BlockSpec((BN, H), lambda i: (i, 0)),
        out_shape=jax.ShapeDtypeStruct((NP, H), F32),
    )(h, f1, f2, f1p, q2, dinvf, *ws, b31, w4p, b4p)


def kernel(in_feat, src, dst, W1, b1, W2, b2, W3, b3, W3_1, b3_1, W4, b4, Wl1, bl1, Wl2, bl2):
    w30, w31, w32 = W3[0:H], W3[H:2 * H], W3[2 * H:3 * H]
    w31s = [W3_1[k * H:(k + 1) * H] for k in range(6)]
    Wla, Wlb, Wlc = Wl1[0:H], Wl1[H:2 * H], Wl1[2 * H:3 * H]

    x_p = jnp.zeros((NP, H), F32).at[:N].set(in_feat)
    pad_i = jnp.arange(EP - E, dtype=jnp.int32)
    src2d = jnp.concatenate([src, pad_i % N]).reshape(RP, 128)
    dst2d = jnp.concatenate([dst, N + (pad_i % (NP - N))]).reshape(RP, 128)

    bias_u = 0.0001
    eps = (bias_u - (1.0 - bias_u)) * jax.random.uniform(jax.random.key(123), (E,), F32) + (1.0 - bias_u)
    leps = jnp.log(eps) - jnp.log(1.0 - eps)
    leps2d = jnp.concatenate([leps, jnp.zeros((EP - E,), F32)]).reshape(RP, 128)

    zrows = jnp.zeros((NP, H), F32)
    z1d = jnp.zeros((NP,), F32)
    ones2d = jnp.ones((RP, 128), F32)

    b3r = b3.reshape(1, H)
    b31r = b3_1.reshape(1, H)
    bl1r = bl1.reshape(1, H)
    wl2row = Wl2.reshape(1, H)
    bl2m = bl2.reshape(1, 1)
    w4p = jnp.zeros((H, H), F32).at[:, :C].set(W4)
    b4p = jnp.zeros((1, H), F32).at[0, :C].set(b4)

    h = _tc_mlp(x_p, W1, b1.reshape(1, H), W2, b2.reshape(1, H))
    deg1p = _seg_scalar(ones2d, dst2d, z1d)

    g0, dinv1f = _tc_scale(h, deg1p)
    p1 = _seg_rows(g0, src2d, dst2d, zrows)
    f1, g1 = _tc_fstep(h, p1, dinv1f)
    p2 = _seg_rows(g1, src2d, dst2d, zrows)
    f2, h1, musum = _tc_big(h, f1, p2, dinv1f, w30, w31, w32, b3r)

    a3, b3d = _gather_pair(h1, h1, src2d, dst2d)
    m2 = _tc_edge(a3.reshape(EP, H), b3d.reshape(EP, H), musum, leps2d,
                  Wla, Wlb, Wlc, bl1r, wl2row, bl2m)

    deg2p = _seg_scalar(m2, dst2d, z1d)
    g0p, dinv2f = _tc_scale(h, deg2p)

    q1 = _seg_masked(g0p, src2d, dst2d, m2, zrows)
    f1p, g1p = _tc_fstep(h, q1, dinv2f)
    q2 = _seg_masked(g1p, src2d, dst2d, m2, zrows)

    out = _tc_final(h, f1, f2, f1p, q2, dinv2f, w31s, b31r, w4p, b4p)
    return out[:N, :C]


# speedup vs baseline: 2.9330x; 1.0953x over previous
import functools

import jax
import jax.numpy as jnp
from jax import lax
from jax.experimental import pallas as pl
from jax.experimental.pallas import tpu as pltpu
from jax.experimental.pallas import tpu_sc as plsc

N = 10000
E = 320000
H = 128
C = 2
THETAS = ((3.0, -3.0, 0.75), (0.0, 3.0, -1.5), (0.0, 0.0, 0.75))

NC, NS, L = 2, 16, 16
NP = 10240
RPT = 80
RP = NC * NS * RPT
EP = RP * 128
NPT = NP // NS
BN = 512
F32 = jnp.float32


def _sc_mesh():
    return plsc.VectorSubcoreMesh(core_axis_name="c", subcore_axis_name="s",
                                  num_cores=NC, num_subcores=NS)


def _seg_scalar_body(vals_hbm, dst_hbm, zeros_hbm, out_hbm, ixd_all, val_all, acc, sem):
    cid = lax.axis_index("c")
    sid = lax.axis_index("s")
    pltpu.sync_copy(zeros_hbm.at[pl.ds(sid * NPT, NPT)], acc.at[pl.ds(sid * NPT, NPT)])
    base = (cid * NS + sid) * RPT
    pltpu.sync_copy(dst_hbm.at[pl.ds(base, RPT)], ixd_all)
    pltpu.sync_copy(vals_hbm.at[pl.ds(base, RPT)], val_all)
    plsc.subcore_barrier()

    def group(g, carry):
        for k in range(16):
            j = g * 16 + k
            pltpu.async_copy(val_all.at[j], acc.at[ixd_all.at[j]], sem, add=True)
        for k in range(16):
            j = g * 16 + k
            pltpu.make_async_copy(val_all.at[j], acc.at[ixd_all.at[j]], sem).wait()
        return carry

    lax.fori_loop(0, RPT // 16, group, 0)
    plsc.subcore_barrier()
    pltpu.sync_copy(acc.at[pl.ds(sid * NPT, NPT)], out_hbm.at[cid, pl.ds(sid * NPT, NPT)])


@functools.cache
def _seg_scalar_kernel():
    return pl.kernel(
        _seg_scalar_body,
        out_type=jax.ShapeDtypeStruct((NC, NP), F32),
        mesh=_sc_mesh(),
        scratch_types=[
            pltpu.VMEM((RPT, 128), jnp.int32),
            pltpu.VMEM((RPT, 128), F32),
            pltpu.VMEM_SHARED((NP,), F32),
            pltpu.SemaphoreType.DMA,
        ],
    )


def _seg_scalar(vals2d, dst2d, z1d):
    return _seg_scalar_kernel()(vals2d, dst2d, z1d)


def _seg_rows_body(g_hbm, src_hbm, dst_hbm, zeros_hbm, out_hbm,
                   ixs0, ixs1, ixd, rows0, rows1, sg0, sg1, acc):
    cid = lax.axis_index("c")
    sid = lax.axis_index("s")
    pltpu.sync_copy(zeros_hbm.at[pl.ds(sid * NPT, NPT)], acc.at[pl.ds(sid * NPT, NPT)])
    base = (cid * NS + sid) * RPT
    plsc.subcore_barrier()

    pltpu.sync_copy(src_hbm.at[base], ixs0)
    pltpu.async_copy(g_hbm.at[ixs0], rows0, sg0)

    def it(j, carry):
        def step(ixa, ixb, rb, ob, sg, sg_n):
            @pl.when(j + 1 < RPT)
            def _():
                pltpu.sync_copy(src_hbm.at[base + j + 1], ixb)
                pltpu.async_copy(g_hbm.at[ixb], ob, sg_n)

            pltpu.sync_copy(dst_hbm.at[base + j], ixd)
            pltpu.make_async_copy(g_hbm.at[ixa], rb, sg).wait()
            pltpu.sync_copy(rb, acc.at[ixd], add=True)

        @pl.when(j % 2 == 0)
        def _():
            step(ixs0, ixs1, rows0, rows1, sg0, sg1)

        @pl.when(j % 2 == 1)
        def _():
            step(ixs1, ixs0, rows1, rows0, sg1, sg0)

        return carry

    lax.fori_loop(0, RPT, it, 0)
    plsc.subcore_barrier()
    pltpu.sync_copy(acc.at[pl.ds(sid * NPT, NPT)], out_hbm.at[cid, pl.ds(sid * NPT, NPT)])


@functools.cache
def _seg_rows_kernel():
    return pl.kernel(
        _seg_rows_body,
        out_type=jax.ShapeDtypeStruct((NC, NP, H), F32),
        mesh=_sc_mesh(),
        scratch_types=[
            pltpu.VMEM((128,), jnp.int32),
            pltpu.VMEM((128,), jnp.int32),
            pltpu.VMEM((128,), jnp.int32),
            pltpu.VMEM((128, H), F32),
            pltpu.VMEM((128, H), F32),
            pltpu.SemaphoreType.DMA,
            pltpu.SemaphoreType.DMA,
            pltpu.VMEM_SHARED((NP, H), F32),
        ],
    )


def _seg_rows(g, src2d, dst2d, zrows):
    return _seg_rows_kernel()(g, src2d, dst2d, zrows)


def _gather_pair_body(g1_hbm, g2_hbm, src_hbm, dst_hbm, a_hbm, b_hbm,
                      ixs0, ixs1, ixd0, ixd1, ra0, ra1, rb0, rb1,
                      sa0, sa1, sb0, sb1):
    cid = lax.axis_index("c")
    sid = lax.axis_index("s")
    base = (cid * NS + sid) * RPT

    pltpu.sync_copy(src_hbm.at[base], ixs0)
    pltpu.sync_copy(dst_hbm.at[base], ixd0)
    pltpu.async_copy(g1_hbm.at[ixs0], ra0, sa0)
    pltpu.async_copy(g2_hbm.at[ixd0], rb0, sb0)

    def it(j, carry):
        def step(ixsa, ixsb, ixda, ixdb, raa, rab, rba, rbb, sga, sga_n, sgb, sgb_n):
            @pl.when(j + 1 < RPT)
            def _():
                pltpu.sync_copy(src_hbm.at[base + j + 1], ixsb)
                pltpu.sync_copy(dst_hbm.at[base + j + 1], ixdb)
                pltpu.async_copy(g1_hbm.at[ixsb], rab, sga_n)
                pltpu.async_copy(g2_hbm.at[ixdb], rbb, sgb_n)

            pltpu.make_async_copy(g1_hbm.at[ixsa], raa, sga).wait()
            pltpu.sync_copy(raa, a_hbm.at[base + j])
            pltpu.make_async_copy(g2_hbm.at[ixda], rba, sgb).wait()
            pltpu.sync_copy(rba, b_hbm.at[base + j])

        @pl.when(j % 2 == 0)
        def _():
            step(ixs0, ixs1, ixd0, ixd1, ra0, ra1, rb0, rb1, sa0, sa1, sb0, sb1)

        @pl.when(j % 2 == 1)
        def _():
            step(ixs1, ixs0, ixd1, ixd0, ra1, ra0, rb1, rb0, sa1, sa0, sb1, sb0)

        return carry

    lax.fori_loop(0, RPT, it, 0)


@functools.cache
def _gather_pair_kernel():
    return pl.kernel(
        _gather_pair_body,
        out_type=(
            jax.ShapeDtypeStruct((RP, 128, H), F32),
            jax.ShapeDtypeStruct((RP, 128, H), F32),
        ),
        mesh=_sc_mesh(),
        scratch_types=[
            pltpu.VMEM((128,), jnp.int32),
            pltpu.VMEM((128,), jnp.int32),
            pltpu.VMEM((128,), jnp.int32),
            pltpu.VMEM((128,), jnp.int32),
            pltpu.VMEM((128, H), F32),
            pltpu.VMEM((128, H), F32),
            pltpu.VMEM((128, H), F32),
            pltpu.VMEM((128, H), F32),
            pltpu.SemaphoreType.DMA,
            pltpu.SemaphoreType.DMA,
            pltpu.SemaphoreType.DMA,
            pltpu.SemaphoreType.DMA,
        ],
    )


def _gather_pair(G1, G2, src2d, dst2d):
    return _gather_pair_kernel()(G1, G2, src2d, dst2d)


def _seg_masked_body(g_hbm, src_hbm, dst_hbm, m2_hbm, zeros_hbm, out_hbm,
                     ixs2, ixd, m2v, rows0, rows1, sg0, sg1, acc):
    cid = lax.axis_index("c")
    sid = lax.axis_index("s")
    pltpu.sync_copy(zeros_hbm.at[pl.ds(sid * NPT, NPT)], acc.at[pl.ds(sid * NPT, NPT)])
    base = (cid * NS + sid) * RPT
    plsc.subcore_barrier()

    pltpu.sync_copy(src_hbm.at[base], ixs2.at[0])
    pltpu.async_copy(g_hbm.at[ixs2.at[0]], rows0, sg0)

    def it(j, carry):
        def step(pa, pb, rb, ob, sg, sg_n):
            @pl.when(j + 1 < RPT)
            def _():
                pltpu.sync_copy(src_hbm.at[base + j + 1], ixs2.at[pb])
                pltpu.async_copy(g_hbm.at[ixs2.at[pb]], ob, sg_n)

            pltpu.sync_copy(dst_hbm.at[base + j], ixd)
            pltpu.sync_copy(m2_hbm.at[base + j], m2v)
            pltpu.make_async_copy(g_hbm.at[ixs2.at[pa]], rb, sg).wait()

            def gbody(g, carry2):
                mv = m2v[pl.ds(g * L, L)]
                for lane in range(L):
                    e = g * L + lane
                    mk = mv[lane]
                    for k in range(8):
                        sl = pl.ds(16 * k, 16)
                        rb[e, sl] = rb[e, sl] * mk
                return carry2

            lax.fori_loop(0, 8, gbody, 0)
            pltpu.sync_copy(rb, acc.at[ixd], add=True)

        @pl.when(j % 2 == 0)
        def _():
            step(0, 1, rows0, rows1, sg0, sg1)

        @pl.when(j % 2 == 1)
        def _():
            step(1, 0, rows1, rows0, sg1, sg0)

        return carry

    lax.fori_loop(0, RPT, it, 0)
    plsc.subcore_barrier()
    pltpu.sync_copy(acc.at[pl.ds(sid * NPT, NPT)], out_hbm.at[cid, pl.ds(sid * NPT, NPT)])


@functools.cache
def _seg_masked_kernel():
    return pl.kernel(
        _seg_masked_body,
        out_type=jax.ShapeDtypeStruct((NC, NP, H), F32),
        mesh=_sc_mesh(),
        scratch_types=[
            pltpu.VMEM((2, 128), jnp.int32),
            pltpu.VMEM((128,), jnp.int32),
            pltpu.VMEM((128,), F32),
            pltpu.VMEM((128, H), F32),
            pltpu.VMEM((128, H), F32),
            pltpu.SemaphoreType.DMA,
            pltpu.SemaphoreType.DMA,
            pltpu.VMEM_SHARED((NP, H), F32),
        ],
    )


def _seg_masked(g, src2d, dst2d, m2, zrows):
    return _seg_masked_kernel()(g, src2d, dst2d, m2, zrows)


_NBLK = NP // BN
_EBLK = EP // BN


def _mlp_body(x_ref, w1_ref, b1_ref, w2_ref, b2_ref, o_ref):
    h = jnp.maximum(jnp.dot(x_ref[...], w1_ref[...], preferred_element_type=F32) + b1_ref[...], 0.0)
    o_ref[...] = jnp.maximum(jnp.dot(h, w2_ref[...], preferred_element_type=F32) + b2_ref[...], 0.0)


def _tc_mlp(x, w1, b1, w2, b2):
    return pl.pallas_call(
        _mlp_body,
        grid=(_NBLK,),
        in_specs=[
            pl.BlockSpec((BN, H), lambda i: (i, 0)),
            pl.BlockSpec((H, H), lambda i: (0, 0)),
            pl.BlockSpec((1, H), lambda i: (0, 0)),
            pl.BlockSpec((H, H), lambda i: (0, 0)),
            pl.BlockSpec((1, H), lambda i: (0, 0)),
        ],
        out_specs=pl.BlockSpec((BN, H), lambda i: (i, 0)),
        out_shape=jax.ShapeDtypeStruct((NP, H), F32),
    )(x, w1, b1, w2, b2)


def _scale_body(h_ref, degp_ref, g_ref, dinv_ref):
    deg = degp_ref[0, :] + degp_ref[1, :]
    dinv = jnp.power(jnp.maximum(deg, 1.0), -0.5)
    dinvf = jnp.broadcast_to(dinv[:, None], (BN, H))
    dinv_ref[...] = dinvf
    g_ref[...] = h_ref[...] * dinvf


def _tc_scale(h, degp):
    return pl.pallas_call(
        _scale_body,
        grid=(_NBLK,),
        in_specs=[
            pl.BlockSpec((BN, H), lambda i: (i, 0)),
            pl.BlockSpec((2, BN), lambda i: (0, i)),
        ],
        out_specs=[
            pl.BlockSpec((BN, H), lambda i: (i, 0)),
            pl.BlockSpec((BN, H), lambda i: (i, 0)),
        ],
        out_shape=[
            jax.ShapeDtypeStruct((NP, H), F32),
            jax.ShapeDtypeStruct((NP, H), F32),
        ],
    )(h, degp)


def _fstep_body(base_ref, p_ref, dinv_ref, f_ref, g_ref):
    ps = p_ref[0] + p_ref[1]
    dinv = dinv_ref[...]
    f = base_ref[...] - ps * dinv
    f_ref[...] = f
    g_ref[...] = f * dinv


def _tc_fstep(base, p, dinvf):
    return pl.pallas_call(
        _fstep_body,
        grid=(_NBLK,),
        in_specs=[
            pl.BlockSpec((BN, H), lambda i: (i, 0)),
            pl.BlockSpec((2, BN, H), lambda i: (0, i, 0)),
            pl.BlockSpec((BN, H), lambda i: (i, 0)),
        ],
        out_specs=[
            pl.BlockSpec((BN, H), lambda i: (i, 0)),
            pl.BlockSpec((BN, H), lambda i: (i, 0)),
        ],
        out_shape=[
            jax.ShapeDtypeStruct((NP, H), F32),
            jax.ShapeDtypeStruct((NP, H), F32),
        ],
    )(base, p, dinvf)


def _big_body(h_ref, f1_ref, p_ref, dinv_ref, w30_ref, w31_ref, w32_ref, bias_ref,
              f2_ref, h1_ref, mu_ref):
    i = pl.program_id(0)
    dinv = dinv_ref[...]
    h = h_ref[...]
    f1 = f1_ref[...]
    f2 = f1 - (p_ref[0] + p_ref[1]) * dinv
    f2_ref[...] = f2
    hs0 = (3.0 * h + (-3.0) * f1) + 0.75 * f2
    hs1 = (0.0 * h + 3.0 * f1) + (-1.5) * f2
    hs2 = (0.0 * h + 0.0 * f1) + 0.75 * f2
    h1 = ((jnp.dot(hs0, w30_ref[...], preferred_element_type=F32)
           + jnp.dot(hs1, w31_ref[...], preferred_element_type=F32))
          + jnp.dot(hs2, w32_ref[...], preferred_element_type=F32)) + bias_ref[...]
    h1_ref[...] = h1
    rowid = i * BN + lax.broadcasted_iota(jnp.int32, (BN, 1), 0)
    h1m = jnp.where(rowid < N, h1, 0.0)
    part = jnp.sum(h1m.reshape(BN // 8, 8, H), axis=0)

    @pl.when(i == 0)
    def _():
        mu_ref[...] = jnp.zeros((8, H), F32)

    mu_ref[...] += part


def _tc_big(h, f1, p2, dinvf, w30, w31, w32, bias3):
    full = lambda shape: pl.BlockSpec(shape, lambda i: tuple(0 for _ in shape))
    return pl.pallas_call(
        _big_body,
        grid=(_NBLK,),
        in_specs=[
            pl.BlockSpec((BN, H), lambda i: (i, 0)),
            pl.BlockSpec((BN, H), lambda i: (i, 0)),
            pl.BlockSpec((2, BN, H), lambda i: (0, i, 0)),
            pl.BlockSpec((BN, H), lambda i: (i, 0)),
            full((H, H)), full((H, H)), full((H, H)), full((1, H)),
        ],
        out_specs=[
            pl.BlockSpec((BN, H), lambda i: (i, 0)),
            pl.BlockSpec((BN, H), lambda i: (i, 0)),
            pl.BlockSpec((8, H), lambda i: (0, 0)),
        ],
        out_shape=[
            jax.ShapeDtypeStruct((NP, H), F32),
            jax.ShapeDtypeStruct((NP, H), F32),
            jax.ShapeDtypeStruct((8, H), F32),
        ],
    )(h, f1, p2, dinvf, w30, w31, w32, bias3)


BE = 1024


def _edge_body(a_ref, b_ref, mu_ref, leps_ref, wla_ref, wlb_ref, wlc_ref,
               bl1_ref, wl2_ref, bl2_ref, m2_ref):
    i = pl.program_id(0)
    mu = jnp.sum(mu_ref[...], axis=0, keepdims=True) / float(N)
    a = a_ref[...]
    b = b_ref[...]
    ef = (((jnp.dot(a - b, wla_ref[...], preferred_element_type=F32)
            + jnp.dot(a - mu, wlb_ref[...], preferred_element_type=F32))
           + jnp.dot(b - mu, wlc_ref[...], preferred_element_type=F32))
          + bl1_ref[...])
    ef = jnp.where(ef > 0, ef, 0.01 * ef)
    s = jnp.sum(ef * wl2_ref[...], axis=1) + bl2_ref[0, 0]
    ef2 = jax.nn.sigmoid(s)
    leps = leps_ref[...].reshape(BE)
    m2 = jax.nn.sigmoid((leps + ef2) / 0.1)
    eid = i * BE + lax.iota(jnp.int32, BE)
    m2 = jnp.where(eid < E, m2, 0.0)
    m2_ref[...] = m2.reshape(BE // 128, 128)


def _tc_edge(a, b, musum, leps2d, wla, wlb, wlc, bl1r, wl2row, bl2m):
    full = lambda shape: pl.BlockSpec(shape, lambda i: tuple(0 for _ in shape))
    return pl.pallas_call(
        _edge_body,
        grid=(EP // BE,),
        in_specs=[
            pl.BlockSpec((BE, H), lambda i: (i, 0)),
            pl.BlockSpec((BE, H), lambda i: (i, 0)),
            full((8, H)),
            pl.BlockSpec((BE // 128, 128), lambda i: (i, 0)),
            full((H, H)), full((H, H)), full((H, H)),
            full((1, H)), full((1, H)), full((1, 1)),
        ],
        out_specs=pl.BlockSpec((BE // 128, 128), lambda i: (i, 0)),
        out_shape=jax.ShapeDtypeStruct((RP, 128), F32),
    )(a, b, musum, leps2d, wla, wlb, wlc, bl1r, wl2row, bl2m)


def _final_body(h_ref, f1_ref, f2_ref, f1p_ref, q_ref, dinv_ref,
                w0_ref, w1_ref, w2_ref, w3_ref, w4w_ref, w5_ref,
                b31_ref, w4_ref, b4_ref, o_ref):
    h = h_ref[...]
    f1 = f1_ref[...]
    f2 = f2_ref[...]
    f1p = f1p_ref[...]
    f2p = f1p - (q_ref[0] + q_ref[1]) * dinv_ref[...]
    hs0 = (3.0 * h + (-3.0) * f1) + 0.75 * f2
    hs1 = (0.0 * h + 3.0 * f1) + (-1.5) * f2
    hs2 = (0.0 * h + 0.0 * f1) + 0.75 * f2
    hp0 = (3.0 * h + (-3.0) * f1p) + 0.75 * f2p
    hp1 = (0.0 * h + 3.0 * f1p) + (-1.5) * f2p
    hp2 = (0.0 * h + 0.0 * f1p) + 0.75 * f2p
    h2 = (((((jnp.dot(hs0, w0_ref[...], preferred_element_type=F32)
              + jnp.dot(hs1, w1_ref[...], preferred_element_type=F32))
             + jnp.dot(hs2, w2_ref[...], preferred_element_type=F32))
            + jnp.dot(hp0, w3_ref[...], preferred_element_type=F32))
           + jnp.dot(hp1, w4w_ref[...], preferred_element_type=F32))
          + jnp.dot(hp2, w5_ref[...], preferred_element_type=F32)) + b31_ref[...]
    h2 = jnp.maximum(h2, 0.0)
    o_ref[...] = jnp.dot(h2, w4_ref[...], preferred_element_type=F32) + b4_ref[...]


def _tc_final(h, f1, f2, f1p, q2, dinvf, ws, b31, w4p, b4p):
    full = lambda shape: pl.BlockSpec(shape, lambda i: tuple(0 for _ in shape))
    return pl.pallas_call(
        _final_body,
        grid=(_NBLK,),
        in_specs=[
            pl.BlockSpec((BN, H), lambda i: (i, 0)),
            pl.BlockSpec((BN, H), lambda i: (i, 0)),
            pl.BlockSpec((BN, H), lambda i: (i, 0)),
            pl.BlockSpec((BN, H), lambda i: (i, 0)),
            pl.BlockSpec((2, BN, H), lambda i: (0, i, 0)),
            pl.BlockSpec((BN, H), lambda i: (i, 0)),
            full((H, H)), full((H, H)), full((H, H)),
            full((H, H)), full((H, H)), full((H, H)),
            full((1, H)), full((H, H)), full((1, H)),
        ],
        out_specs=pl.BlockSpec((BN, H), lambda i: (i, 0)),
        out_shape=jax.ShapeDtypeStruct((NP, H), F32),
    )(h, f1, f2, f1p, q2, dinvf, *ws, b31, w4p, b4p)


def kernel(in_feat, src, dst, W1, b1, W2, b2, W3, b3, W3_1, b3_1, W4, b4, Wl1, bl1, Wl2, bl2):
    w30, w31, w32 = W3[0:H], W3[H:2 * H], W3[2 * H:3 * H]
    w31s = [W3_1[k * H:(k + 1) * H] for k in range(6)]
    Wla, Wlb, Wlc = Wl1[0:H], Wl1[H:2 * H], Wl1[2 * H:3 * H]

    x_p = jnp.zeros((NP, H), F32).at[:N].set(in_feat)
    pad_i = jnp.arange(EP - E, dtype=jnp.int32)
    src2d = jnp.concatenate([src, pad_i % N]).reshape(RP, 128)
    dst2d = jnp.concatenate([dst, N + (pad_i % (NP - N))]).reshape(RP, 128)

    bias_u = 0.0001
    eps = (bias_u - (1.0 - bias_u)) * jax.random.uniform(jax.random.key(123), (E,), F32) + (1.0 - bias_u)
    leps = jnp.log(eps) - jnp.log(1.0 - eps)
    leps2d = jnp.concatenate([leps, jnp.zeros((EP - E,), F32)]).reshape(RP, 128)

    zrows = jnp.zeros((NP, H), F32)
    z1d = jnp.zeros((NP,), F32)
    ones2d = jnp.ones((RP, 128), F32)

    b3r = b3.reshape(1, H)
    b31r = b3_1.reshape(1, H)
    bl1r = bl1.reshape(1, H)
    wl2row = Wl2.reshape(1, H)
    bl2m = bl2.reshape(1, 1)
    w4p = jnp.zeros((H, H), F32).at[:, :C].set(W4)
    b4p = jnp.zeros((1, H), F32).at[0, :C].set(b4)

    h = _tc_mlp(x_p, W1, b1.reshape(1, H), W2, b2.reshape(1, H))
    deg1p = _seg_scalar(ones2d, dst2d, z1d)

    g0, dinv1f = _tc_scale(h, deg1p)
    p1 = _seg_rows(g0, src2d, dst2d, zrows)
    f1, g1 = _tc_fstep(h, p1, dinv1f)
    p2 = _seg_rows(g1, src2d, dst2d, zrows)
    f2, h1, musum = _tc_big(h, f1, p2, dinv1f, w30, w31, w32, b3r)

    a3, b3d = _gather_pair(h1, h1, src2d, dst2d)
    m2 = _tc_edge(a3.reshape(EP, H), b3d.reshape(EP, H), musum, leps2d,
                  Wla, Wlb, Wlc, bl1r, wl2row, bl2m)

    deg2p = _seg_scalar(m2, dst2d, z1d)
    g0p, dinv2f = _tc_scale(h, deg2p)

    q1 = _seg_masked(g0p, src2d, dst2d, m2, zrows)
    f1p, g1p = _tc_fstep(h, q1, dinv2f)
    q2 = _seg_masked(g1p, src2d, dst2d, m2, zrows)

    out = _tc_final(h, f1, f2, f1p, q2, dinv2f, w31s, b31r, w4p, b4p)
    return out[:N, :C]
